# Initial kernel scaffold; baseline (speedup 1.0000x reference)
import jax
import jax.numpy as jnp
from jax.experimental import pallas as pl


def kernel(x_target, x_reference, edge_index_tt, edge_index_rr, edge_index_rt, params):
    raise NotImplementedError("write your pallas kernel here")


# speedup vs baseline: 1.6617x; 1.6617x over previous
import functools

import jax
import jax.numpy as jnp
from jax import lax
from jax.experimental import pallas as pl
from jax.experimental.pallas import tpu as pltpu
from jax.experimental.pallas import tpu_sc as plsc

N = 10000
NPAD = 10240
RPT = NPAD // 16
NB = 5000
NH = 6000
RPTH = 376
NS = 2 * NH
ECH = 1280
CPT = ECH // 16
F = 128
CW = 16
HID = 256


def _leaky(x):
    return jnp.where(x >= 0.0, x, 0.2 * x)


@functools.cache
def _sc_scatter3_kernel():
    mesh = plsc.VectorSubcoreMesh(core_axis_name="c", subcore_axis_name="s")
    ssd = jax.ShapeDtypeStruct((2, NS, F), jnp.float32)
    ECT = CPT * 128

    @functools.partial(
        pl.kernel,
        out_type=(ssd, ssd, ssd),
        mesh=mesh,
        scratch_types=[
            pltpu.VMEM((ECT,), jnp.int32),
            pltpu.VMEM((8, 128), jnp.int32),
            pltpu.VMEM((128, F), jnp.float32),
            pltpu.VMEM_SHARED((NH, F), jnp.float32),
            pltpu.SemaphoreType.DMA,
        ],
    )
    def k(tlo, thi, rlo, rhi, stt, dtt0, dtt1, srr, drr0, drr1,
          srt, drt0, drt1, zz, out_tt, out_rr, out_rt,
          src_v, dst8, rows_v, acc, sem):
        c = lax.axis_index("c")
        s = lax.axis_index("s")
        r0 = s * RPTH

        def sliced(fn):
            @pl.when(s < 15)
            def _():
                fn(r0, RPTH)

            @pl.when(s == 15)
            def _():
                fn(15 * RPTH, NH - 15 * RPTH)

        def phase(tab0, tab1, srcm, dstm, out, h):
            sliced(lambda r, n: pltpu.sync_copy(zz.at[pl.ds(0, n)],
                                                acc.at[pl.ds(r, n)]))
            pltpu.sync_copy(srcm.at[pl.ds(s * ECT, ECT)], src_v)
            plsc.subcore_barrier()

            def run(tab):
                def body(g, carry):
                    pltpu.sync_copy(dstm.at[s * (CPT // 8) + g], dst8)

                    def inner(jj, c2):
                        soff = pl.multiple_of((g * 8 + jj) * 128, 8)
                        pltpu.async_copy(
                            tab.at[src_v.at[pl.ds(soff, 128)]], rows_v, sem
                        ).wait()
                        pltpu.sync_copy(rows_v, acc.at[dst8.at[jj]], add=True)
                        return c2
                    lax.fori_loop(0, 8, inner, 0)
                    return carry
                lax.fori_loop(0, CPT // 8, body, 0)

            @pl.when(c == 0)
            def _():
                run(tab0)

            @pl.when(c == 1)
            def _():
                run(tab1)

            plsc.subcore_barrier()
            sliced(lambda r, n: pltpu.sync_copy(
                acc.at[pl.ds(r, n)], out.at[c, pl.ds(h * NH + r, n)]))
            plsc.subcore_barrier()

        phase(tlo, thi, stt, dtt0, out_tt, 0)
        phase(tlo, thi, stt, dtt1, out_tt, 1)
        phase(rlo, rhi, srr, drr0, out_rr, 0)
        phase(rlo, rhi, srr, drr1, out_rr, 1)
        phase(rlo, rhi, srt, drt0, out_rt, 0)
        phase(rlo, rhi, srt, drt1, out_rt, 1)

    return k


def _tc_post(x, W1, Wp, g, b):
    nrows, kdim = x.shape
    R = 1000

    def body(x_ref, w1_ref, wp_ref, g_ref, b_ref, lo_ref, hi_ref):
        h = jnp.dot(x_ref[...], w1_ref[...], preferred_element_type=jnp.float32)
        h = _leaky(h)
        h = jnp.dot(h, wp_ref[...], preferred_element_type=jnp.float32)
        m = jnp.mean(h, axis=1, keepdims=True)
        v = jnp.mean((h - m) * (h - m), axis=1, keepdims=True)
        h = (h - m) * lax.rsqrt(v + 1e-5) * g_ref[...] + b_ref[...]
        h = _leaky(h)
        lo_ref[...] = h[:, :F]
        hi_ref[...] = h[:, F:]

    return pl.pallas_call(
        body,
        grid=(nrows // R,),
        in_specs=[
            pl.BlockSpec((R, kdim), lambda i: (i, 0)),
            pl.BlockSpec((kdim, HID), lambda i: (0, 0)),
            pl.BlockSpec((HID, HID), lambda i: (0, 0)),
            pl.BlockSpec((1, HID), lambda i: (0, 0)),
            pl.BlockSpec((1, HID), lambda i: (0, 0)),
        ],
        out_specs=[
            pl.BlockSpec((R, F), lambda i: (i, 0)),
            pl.BlockSpec((R, F), lambda i: (i, 0)),
        ],
        out_shape=[
            jax.ShapeDtypeStruct((nrows, F), jnp.float32),
            jax.ShapeDtypeStruct((nrows, F), jnp.float32),
        ],
    )(x, W1, Wp, g, b)


def _tc_combine_tgt(s_tt, s_rt, cnt_tt, cnt_rt, h_lo, h_hi,
                    wl_tt, wr_tt, b_tt, wl_rt, wr_rt, b_rt, final):
    R = 1000

    def body(stt_ref, srt_ref, ctt_ref, crt_ref, hlo_ref, hhi_ref,
             wltt_ref, wrtt_ref, btt_ref, wlrt_ref, wrrt_ref, brt_ref, *outs):
        ctt = jnp.maximum(ctt_ref[0], 1.0)
        crt = jnp.maximum(crt_ref[0], 1.0)
        y = jnp.dot(stt_ref[0] / ctt, wltt_ref[0:F, :],
                    preferred_element_type=jnp.float32)
        y = y + jnp.dot(stt_ref[1] / ctt, wltt_ref[F:, :],
                        preferred_element_type=jnp.float32)
        y = y + jnp.dot(srt_ref[0] / crt, wlrt_ref[0:F, :],
                        preferred_element_type=jnp.float32)
        y = y + jnp.dot(srt_ref[1] / crt, wlrt_ref[F:, :],
                        preferred_element_type=jnp.float32)
        y = y + jnp.dot(hlo_ref[...], wrtt_ref[0:F, :] + wrrt_ref[0:F, :],
                        preferred_element_type=jnp.float32)
        y = y + jnp.dot(hhi_ref[...], wrtt_ref[F:, :] + wrrt_ref[F:, :],
                        preferred_element_type=jnp.float32)
        y = y + btt_ref[...] + brt_ref[...]
        h = _leaky(0.5 * y)
        if final:
            outs[0][...] = h
        else:
            outs[0][...] = h[:, :F]
            outs[1][...] = h[:, F:]

    if final:
        out_specs = [pl.BlockSpec((R, HID), lambda i: (i, 0))]
        out_shape = [jax.ShapeDtypeStruct((N, HID), jnp.float32)]
    else:
        out_specs = [pl.BlockSpec((R, F), lambda i: (i, 0)),
                     pl.BlockSpec((R, F), lambda i: (i, 0))]
        out_shape = [jax.ShapeDtypeStruct((N, F), jnp.float32),
                     jax.ShapeDtypeStruct((N, F), jnp.float32)]

    smap = lambda i: (0, jnp.where(i < NB // R, i, i + 1), 0)
    return pl.pallas_call(
        body,
        grid=(N // R,),
        in_specs=[
            pl.BlockSpec((2, R, F), smap),
            pl.BlockSpec((2, R, F), smap),
            pl.BlockSpec((1, R, F), smap),
            pl.BlockSpec((1, R, F), smap),
            pl.BlockSpec((R, F), lambda i: (i, 0)),
            pl.BlockSpec((R, F), lambda i: (i, 0)),
            pl.BlockSpec((HID, HID), lambda i: (0, 0)),
            pl.BlockSpec((HID, HID), lambda i: (0, 0)),
            pl.BlockSpec((1, HID), lambda i: (0, 0)),
            pl.BlockSpec((HID, HID), lambda i: (0, 0)),
            pl.BlockSpec((HID, HID), lambda i: (0, 0)),
            pl.BlockSpec((1, HID), lambda i: (0, 0)),
        ],
        out_specs=out_specs,
        out_shape=out_shape,
    )(s_tt, s_rt, cnt_tt, cnt_rt, h_lo, h_hi,
      wl_tt, wr_tt, b_tt, wl_rt, wr_rt, b_rt)


def _tc_combine_ref(s_rr, cnt, h_lo, h_hi, wl_rr, wr_rr, b_rr, final):
    R = 1000

    def body(srr_ref, crr_ref, hlo_ref, hhi_ref,
             wl_ref, wr_ref, b_ref, *outs):
        crr = jnp.maximum(crr_ref[0], 1.0)
        y = jnp.dot(srr_ref[0] / crr, wl_ref[0:F, :],
                    preferred_element_type=jnp.float32)
        y = y + jnp.dot(srr_ref[1] / crr, wl_ref[F:, :],
                        preferred_element_type=jnp.float32)
        y = y + jnp.dot(hlo_ref[...], wr_ref[0:F, :],
                        preferred_element_type=jnp.float32)
        y = y + jnp.dot(hhi_ref[...], wr_ref[F:, :],
                        preferred_element_type=jnp.float32)
        y = y + b_ref[...]
        h = _leaky(y)
        if final:
            outs[0][...] = h
        else:
            outs[0][...] = h[:, :F]
            outs[1][...] = h[:, F:]

    if final:
        out_specs = [pl.BlockSpec((R, HID), lambda i: (i, 0))]
        out_shape = [jax.ShapeDtypeStruct((N, HID), jnp.float32)]
    else:
        out_specs = [pl.BlockSpec((R, F), lambda i: (i, 0)),
                     pl.BlockSpec((R, F), lambda i: (i, 0))]
        out_shape = [jax.ShapeDtypeStruct((N, F), jnp.float32),
                     jax.ShapeDtypeStruct((N, F), jnp.float32)]

    smap = lambda i: (0, jnp.where(i < NB // R, i, i + 1), 0)
    return pl.pallas_call(
        body,
        grid=(N // R,),
        in_specs=[
            pl.BlockSpec((2, R, F), smap),
            pl.BlockSpec((1, R, F), smap),
            pl.BlockSpec((R, F), lambda i: (i, 0)),
            pl.BlockSpec((R, F), lambda i: (i, 0)),
            pl.BlockSpec((HID, HID), lambda i: (0, 0)),
            pl.BlockSpec((HID, HID), lambda i: (0, 0)),
            pl.BlockSpec((1, HID), lambda i: (0, 0)),
        ],
        out_specs=out_specs,
        out_shape=out_shape,
    )(s_rr, cnt, h_lo, h_hi, wl_rr, wr_rr, b_rr)


def _prep_edges(ei):
    e = ei.shape[1]
    pad = ECH * 128 - e
    spread = jnp.arange(pad, dtype=jnp.int32)
    src = jnp.concatenate([ei[0], spread % 4096])
    dst = jnp.concatenate([ei[1], N + (spread & 511)])
    trash = NB + (dst & 511)
    dst_h0 = jnp.where(dst < NB, dst, trash)
    dst_h1 = jnp.where(dst >= NB, dst - NB, trash)
    dst_h1 = jnp.where(dst_h1 >= NB, trash, dst_h1)
    dst_c = jnp.where(dst < N, dst, N + (dst & 127))
    g3 = (ECH // 8, 8, 128)
    return (src, dst_h0.reshape(g3), dst_h1.reshape(g3), dst_c.reshape(g3))


def kernel(x_target, x_reference, edge_index_tt, edge_index_rr, edge_index_rt,
           params):
    p = params
    src_tt, dtt0, dtt1, _ = _prep_edges(edge_index_tt)
    src_rr, drr0, drr1, _ = _prep_edges(edge_index_rr)
    src_rt, drt0, drt1, _ = _prep_edges(edge_index_rt)

    zeros_f = jnp.zeros((RPTH, F), jnp.float32)
    ones_t = jnp.ones((N, F), jnp.float32)

    g2 = p['ln_g'].reshape(1, HID)
    b2 = p['ln_b'].reshape(1, HID)

    ht_lo, ht_hi = _tc_post(x_target, p['W_win'], p['W_post'], g2, b2)
    hr_lo, hr_hi = _tc_post(x_reference, p['W_exp'], p['W_post'], g2, b2)

    cnt_tt, cnt_rr, cnt_rt = _sc_scatter3_kernel()(
        ones_t, ones_t, ones_t, ones_t, src_tt, dtt0, dtt1,
        src_rr, drr0, drr1, src_rt, drt0, drt1, zeros_f)

    names = ('Wl_tt', 'Wr_tt', 'b_tt', 'Wl_rr', 'Wr_rr', 'b_rr',
             'Wl_rt', 'Wr_rt', 'b_rt')
    ws = {n: jnp.stack([layer[n] for layer in p['layers']]) for n in names}

    def step(carry, w):
        ht_lo, ht_hi, hr_lo, hr_hi = carry
        s_tt, s_rr, s_rt = _sc_scatter3_kernel()(
            ht_lo, ht_hi, hr_lo, hr_hi, src_tt, dtt0, dtt1,
            src_rr, drr0, drr1, src_rt, drt0, drt1, zeros_f)
        nt_lo, nt_hi = _tc_combine_tgt(
            s_tt, s_rt, cnt_tt, cnt_rt, ht_lo, ht_hi,
            w['Wl_tt'], w['Wr_tt'], w['b_tt'].reshape(1, HID),
            w['Wl_rt'], w['Wr_rt'], w['b_rt'].reshape(1, HID), False)
        nr_lo, nr_hi = _tc_combine_ref(
            s_rr, cnt_rr, hr_lo, hr_hi,
            w['Wl_rr'], w['Wr_rr'], w['b_rr'].reshape(1, HID), False)
        return (nt_lo, nt_hi, nr_lo, nr_hi), None

    (ht_lo, ht_hi, hr_lo, hr_hi), _ = lax.scan(
        step, (ht_lo, ht_hi, hr_lo, hr_hi), ws)

    h_tgt = jnp.concatenate([ht_lo, ht_hi], axis=1)
    h_ref = jnp.concatenate([hr_lo, hr_hi], axis=1)
    return (h_tgt, h_ref)


# speedup vs baseline: 2.7228x; 1.6385x over previous
import functools

import jax
import jax.numpy as jnp
from jax import lax
from jax.experimental import pallas as pl
from jax.experimental.pallas import tpu as pltpu
from jax.experimental.pallas import tpu_sc as plsc

N = 10000
NPAD = 10240
RPT = NPAD // 16
NB = 5000
NH = 6000
RPTH = 376
NS = 2 * NH
ECH = 1280
CPT = ECH // 16
F = 128
CW = 16
HID = 256


def _leaky(x):
    return jnp.where(x >= 0.0, x, 0.2 * x)


@functools.cache
def _sc_scatter3_kernel():
    mesh = plsc.VectorSubcoreMesh(core_axis_name="c", subcore_axis_name="s")
    ssd = jax.ShapeDtypeStruct((2, NS, F), jnp.float32)
    ECT = CPT * 128

    @functools.partial(
        pl.kernel,
        out_type=(ssd, ssd, ssd),
        mesh=mesh,
        scratch_types=[
            pltpu.VMEM((ECT,), jnp.int32),
            pltpu.VMEM((8, 128), jnp.int32),
            pltpu.VMEM((2, 128, F), jnp.float32),
            pltpu.VMEM_SHARED((NH, F), jnp.float32),
            pltpu.SemaphoreType.DMA((2,)),
        ],
    )
    def k(tlo, thi, rlo, rhi, stt, dtt0, dtt1, srr, drr0, drr1,
          srt, drt0, drt1, zz, out_tt, out_rr, out_rt,
          src_v, dst8, rows_v, acc, sem):
        c = lax.axis_index("c")
        s = lax.axis_index("s")
        r0 = s * RPTH

        def sliced(fn):
            @pl.when(s < 15)
            def _():
                fn(r0, RPTH)

            @pl.when(s == 15)
            def _():
                fn(15 * RPTH, NH - 15 * RPTH)

        def phase(tab0, tab1, srcm, dstm, out, h):
            sliced(lambda r, n: pltpu.sync_copy(zz.at[pl.ds(0, n)],
                                                acc.at[pl.ds(r, n)]))
            pltpu.sync_copy(srcm.at[pl.ds(s * ECT, ECT)], src_v)
            plsc.subcore_barrier()

            def run(tab):
                def gstart(j):
                    soff = pl.multiple_of(j * 128, 8)
                    b = lax.rem(j, 2)
                    pltpu.async_copy(tab.at[src_v.at[pl.ds(soff, 128)]],
                                     rows_v.at[b], sem.at[b])

                gstart(0)

                def body(j, carry):
                    @pl.when(j + 1 < CPT)
                    def _():
                        gstart(j + 1)

                    @pl.when(lax.rem(j, 8) == 0)
                    def _():
                        pltpu.sync_copy(
                            dstm.at[s * (CPT // 8) + lax.div(j, 8)], dst8)
                    b = lax.rem(j, 2)
                    soff = pl.multiple_of(j * 128, 8)
                    pltpu.make_async_copy(
                        tab.at[src_v.at[pl.ds(soff, 128)]],
                        rows_v.at[b], sem.at[b]).wait()
                    pltpu.sync_copy(rows_v.at[b],
                                    acc.at[dst8.at[lax.rem(j, 8)]], add=True)
                    return carry
                lax.fori_loop(0, CPT, body, 0)

            @pl.when(c == 0)
            def _():
                run(tab0)

            @pl.when(c == 1)
            def _():
                run(tab1)

            plsc.subcore_barrier()
            sliced(lambda r, n: pltpu.sync_copy(
                acc.at[pl.ds(r, n)], out.at[c, pl.ds(h * NH + r, n)]))
            plsc.subcore_barrier()

        phase(tlo, thi, stt, dtt0, out_tt, 0)
        phase(tlo, thi, stt, dtt1, out_tt, 1)
        phase(rlo, rhi, srr, drr0, out_rr, 0)
        phase(rlo, rhi, srr, drr1, out_rr, 1)
        phase(rlo, rhi, srt, drt0, out_rt, 0)
        phase(rlo, rhi, srt, drt1, out_rt, 1)

    return k


def _tc_post(x, W1, Wp, g, b):
    nrows, kdim = x.shape
    R = 1000

    def body(x_ref, w1_ref, wp_ref, g_ref, b_ref, lo_ref, hi_ref):
        h = jnp.dot(x_ref[...], w1_ref[...], preferred_element_type=jnp.float32)
        h = _leaky(h)
        h = jnp.dot(h, wp_ref[...], preferred_element_type=jnp.float32)
        m = jnp.mean(h, axis=1, keepdims=True)
        v = jnp.mean((h - m) * (h - m), axis=1, keepdims=True)
        h = (h - m) * lax.rsqrt(v + 1e-5) * g_ref[...] + b_ref[...]
        h = _leaky(h)
        lo_ref[...] = h[:, :F]
        hi_ref[...] = h[:, F:]

    return pl.pallas_call(
        body,
        grid=(nrows // R,),
        in_specs=[
            pl.BlockSpec((R, kdim), lambda i: (i, 0)),
            pl.BlockSpec((kdim, HID), lambda i: (0, 0)),
            pl.BlockSpec((HID, HID), lambda i: (0, 0)),
            pl.BlockSpec((1, HID), lambda i: (0, 0)),
            pl.BlockSpec((1, HID), lambda i: (0, 0)),
        ],
        out_specs=[
            pl.BlockSpec((R, F), lambda i: (i, 0)),
            pl.BlockSpec((R, F), lambda i: (i, 0)),
        ],
        out_shape=[
            jax.ShapeDtypeStruct((nrows, F), jnp.float32),
            jax.ShapeDtypeStruct((nrows, F), jnp.float32),
        ],
    )(x, W1, Wp, g, b)


def _tc_combine_tgt(s_tt, s_rt, cnt_tt, cnt_rt, h_lo, h_hi,
                    wl_tt, wr_tt, b_tt, wl_rt, wr_rt, b_rt, final):
    R = 1000

    def body(stt_ref, srt_ref, ctt_ref, crt_ref, hlo_ref, hhi_ref,
             wltt_ref, wrtt_ref, btt_ref, wlrt_ref, wrrt_ref, brt_ref, *outs):
        ctt = jnp.maximum(ctt_ref[0], 1.0)
        crt = jnp.maximum(crt_ref[0], 1.0)
        y = jnp.dot(stt_ref[0] / ctt, wltt_ref[0:F, :],
                    preferred_element_type=jnp.float32)
        y = y + jnp.dot(stt_ref[1] / ctt, wltt_ref[F:, :],
                        preferred_element_type=jnp.float32)
        y = y + jnp.dot(srt_ref[0] / crt, wlrt_ref[0:F, :],
                        preferred_element_type=jnp.float32)
        y = y + jnp.dot(srt_ref[1] / crt, wlrt_ref[F:, :],
                        preferred_element_type=jnp.float32)
        y = y + jnp.dot(hlo_ref[...], wrtt_ref[0:F, :] + wrrt_ref[0:F, :],
                        preferred_element_type=jnp.float32)
        y = y + jnp.dot(hhi_ref[...], wrtt_ref[F:, :] + wrrt_ref[F:, :],
                        preferred_element_type=jnp.float32)
        y = y + btt_ref[...] + brt_ref[...]
        h = _leaky(0.5 * y)
        if final:
            outs[0][...] = h
        else:
            outs[0][...] = h[:, :F]
            outs[1][...] = h[:, F:]

    if final:
        out_specs = [pl.BlockSpec((R, HID), lambda i: (i, 0))]
        out_shape = [jax.ShapeDtypeStruct((N, HID), jnp.float32)]
    else:
        out_specs = [pl.BlockSpec((R, F), lambda i: (i, 0)),
                     pl.BlockSpec((R, F), lambda i: (i, 0))]
        out_shape = [jax.ShapeDtypeStruct((N, F), jnp.float32),
                     jax.ShapeDtypeStruct((N, F), jnp.float32)]

    smap = lambda i: (0, jnp.where(i < NB // R, i, i + 1), 0)
    return pl.pallas_call(
        body,
        grid=(N // R,),
        in_specs=[
            pl.BlockSpec((2, R, F), smap),
            pl.BlockSpec((2, R, F), smap),
            pl.BlockSpec((1, R, F), smap),
            pl.BlockSpec((1, R, F), smap),
            pl.BlockSpec((R, F), lambda i: (i, 0)),
            pl.BlockSpec((R, F), lambda i: (i, 0)),
            pl.BlockSpec((HID, HID), lambda i: (0, 0)),
            pl.BlockSpec((HID, HID), lambda i: (0, 0)),
            pl.BlockSpec((1, HID), lambda i: (0, 0)),
            pl.BlockSpec((HID, HID), lambda i: (0, 0)),
            pl.BlockSpec((HID, HID), lambda i: (0, 0)),
            pl.BlockSpec((1, HID), lambda i: (0, 0)),
        ],
        out_specs=out_specs,
        out_shape=out_shape,
    )(s_tt, s_rt, cnt_tt, cnt_rt, h_lo, h_hi,
      wl_tt, wr_tt, b_tt, wl_rt, wr_rt, b_rt)


def _tc_combine_ref(s_rr, cnt, h_lo, h_hi, wl_rr, wr_rr, b_rr, final):
    R = 1000

    def body(srr_ref, crr_ref, hlo_ref, hhi_ref,
             wl_ref, wr_ref, b_ref, *outs):
        crr = jnp.maximum(crr_ref[0], 1.0)
        y = jnp.dot(srr_ref[0] / crr, wl_ref[0:F, :],
                    preferred_element_type=jnp.float32)
        y = y + jnp.dot(srr_ref[1] / crr, wl_ref[F:, :],
                        preferred_element_type=jnp.float32)
        y = y + jnp.dot(hlo_ref[...], wr_ref[0:F, :],
                        preferred_element_type=jnp.float32)
        y = y + jnp.dot(hhi_ref[...], wr_ref[F:, :],
                        preferred_element_type=jnp.float32)
        y = y + b_ref[...]
        h = _leaky(y)
        if final:
            outs[0][...] = h
        else:
            outs[0][...] = h[:, :F]
            outs[1][...] = h[:, F:]

    if final:
        out_specs = [pl.BlockSpec((R, HID), lambda i: (i, 0))]
        out_shape = [jax.ShapeDtypeStruct((N, HID), jnp.float32)]
    else:
        out_specs = [pl.BlockSpec((R, F), lambda i: (i, 0)),
                     pl.BlockSpec((R, F), lambda i: (i, 0))]
        out_shape = [jax.ShapeDtypeStruct((N, F), jnp.float32),
                     jax.ShapeDtypeStruct((N, F), jnp.float32)]

    smap = lambda i: (0, jnp.where(i < NB // R, i, i + 1), 0)
    return pl.pallas_call(
        body,
        grid=(N // R,),
        in_specs=[
            pl.BlockSpec((2, R, F), smap),
            pl.BlockSpec((1, R, F), smap),
            pl.BlockSpec((R, F), lambda i: (i, 0)),
            pl.BlockSpec((R, F), lambda i: (i, 0)),
            pl.BlockSpec((HID, HID), lambda i: (0, 0)),
            pl.BlockSpec((HID, HID), lambda i: (0, 0)),
            pl.BlockSpec((1, HID), lambda i: (0, 0)),
        ],
        out_specs=out_specs,
        out_shape=out_shape,
    )(s_rr, cnt, h_lo, h_hi, wl_rr, wr_rr, b_rr)


def _prep_edges(ei):
    e = ei.shape[1]
    pad = ECH * 128 - e
    spread = jnp.arange(pad, dtype=jnp.int32)
    src = jnp.concatenate([ei[0], spread % 4096])
    dst = jnp.concatenate([ei[1], N + (spread & 511)])
    trash = NB + (dst & 511)
    dst_h0 = jnp.where(dst < NB, dst, trash)
    dst_h1 = jnp.where(dst >= NB, dst - NB, trash)
    dst_h1 = jnp.where(dst_h1 >= NB, trash, dst_h1)
    dst_c = jnp.where(dst < N, dst, N + (dst & 127))
    g3 = (ECH // 8, 8, 128)
    return (src, dst_h0.reshape(g3), dst_h1.reshape(g3), dst_c.reshape(g3))


def kernel(x_target, x_reference, edge_index_tt, edge_index_rr, edge_index_rt,
           params):
    p = params
    src_tt, dtt0, dtt1, _ = _prep_edges(edge_index_tt)
    src_rr, drr0, drr1, _ = _prep_edges(edge_index_rr)
    src_rt, drt0, drt1, _ = _prep_edges(edge_index_rt)

    zeros_f = jnp.zeros((RPTH, F), jnp.float32)
    ones_t = jnp.ones((N, F), jnp.float32)

    g2 = p['ln_g'].reshape(1, HID)
    b2 = p['ln_b'].reshape(1, HID)

    ht_lo, ht_hi = _tc_post(x_target, p['W_win'], p['W_post'], g2, b2)
    hr_lo, hr_hi = _tc_post(x_reference, p['W_exp'], p['W_post'], g2, b2)

    cnt_tt, cnt_rr, cnt_rt = _sc_scatter3_kernel()(
        ones_t, ones_t, ones_t, ones_t, src_tt, dtt0, dtt1,
        src_rr, drr0, drr1, src_rt, drt0, drt1, zeros_f)

    names = ('Wl_tt', 'Wr_tt', 'b_tt', 'Wl_rr', 'Wr_rr', 'b_rr',
             'Wl_rt', 'Wr_rt', 'b_rt')
    ws = {n: jnp.stack([layer[n] for layer in p['layers']]) for n in names}

    def step(carry, w):
        ht_lo, ht_hi, hr_lo, hr_hi = carry
        s_tt, s_rr, s_rt = _sc_scatter3_kernel()(
            ht_lo, ht_hi, hr_lo, hr_hi, src_tt, dtt0, dtt1,
            src_rr, drr0, drr1, src_rt, drt0, drt1, zeros_f)
        nt_lo, nt_hi = _tc_combine_tgt(
            s_tt, s_rt, cnt_tt, cnt_rt, ht_lo, ht_hi,
            w['Wl_tt'], w['Wr_tt'], w['b_tt'].reshape(1, HID),
            w['Wl_rt'], w['Wr_rt'], w['b_rt'].reshape(1, HID), False)
        nr_lo, nr_hi = _tc_combine_ref(
            s_rr, cnt_rr, hr_lo, hr_hi,
            w['Wl_rr'], w['Wr_rr'], w['b_rr'].reshape(1, HID), False)
        return (nt_lo, nt_hi, nr_lo, nr_hi), None

    (ht_lo, ht_hi, hr_lo, hr_hi), _ = lax.scan(
        step, (ht_lo, ht_hi, hr_lo, hr_hi), ws)

    h_tgt = jnp.concatenate([ht_lo, ht_hi], axis=1)
    h_ref = jnp.concatenate([hr_lo, hr_hi], axis=1)
    return (h_tgt, h_ref)


# speedup vs baseline: 3.2928x; 1.2094x over previous
import functools

import jax
import jax.numpy as jnp
from jax import lax
from jax.experimental import pallas as pl
from jax.experimental.pallas import tpu as pltpu
from jax.experimental.pallas import tpu_sc as plsc

N = 10000
NPAD = 10240
RPT = NPAD // 16
NB = 5000
NH = 6000
RPTH = 376
NS = 2 * NH
ECH = 1280
CPT = ECH // 16
F = 128
CW = 16
HID = 256


def _leaky(x):
    return jnp.where(x >= 0.0, x, 0.2 * x)


@functools.cache
def _sc_scatter3_kernel():
    mesh = plsc.VectorSubcoreMesh(core_axis_name="c", subcore_axis_name="s")
    ssd = jax.ShapeDtypeStruct((2, NS, F), jnp.float32)
    ECT = CPT * 128

    @functools.partial(
        pl.kernel,
        out_type=(ssd, ssd, ssd),
        mesh=mesh,
        scratch_types=[
            pltpu.VMEM((ECT,), jnp.int32),
            pltpu.VMEM((8, 128), jnp.int32),
            pltpu.VMEM((2, 128, F), jnp.float32),
            pltpu.VMEM_SHARED((NH, F), jnp.float32),
            pltpu.SemaphoreType.DMA((2,)),
        ],
    )
    def k(tlo, thi, rlo, rhi, stt, dtt0, dtt1, srr, drr0, drr1,
          srt, drt0, drt1, zz, out_tt, out_rr, out_rt,
          src_v, dst8, rows_v, acc, sem):
        c = lax.axis_index("c")
        s = lax.axis_index("s")
        r0 = s * RPTH

        def sliced(fn):
            @pl.when(s < 15)
            def _():
                fn(r0, RPTH)

            @pl.when(s == 15)
            def _():
                fn(15 * RPTH, NH - 15 * RPTH)

        def phase(tab0, tab1, srcm, dstm, out, h):
            sliced(lambda r, n: pltpu.sync_copy(zz.at[pl.ds(0, n)],
                                                acc.at[pl.ds(r, n)]))
            pltpu.sync_copy(srcm.at[pl.ds(s * ECT, ECT)], src_v)
            plsc.subcore_barrier()

            def run(tab):
                def gstart(j):
                    soff = pl.multiple_of(j * 128, 8)
                    b = lax.rem(j, 2)
                    pltpu.async_copy(tab.at[src_v.at[pl.ds(soff, 128)]],
                                     rows_v.at[b], sem.at[b])

                gstart(0)

                def body(j, carry):
                    @pl.when(j + 1 < CPT)
                    def _():
                        gstart(j + 1)

                    @pl.when(lax.rem(j, 8) == 0)
                    def _():
                        pltpu.sync_copy(
                            dstm.at[s * (CPT // 8) + lax.div(j, 8)], dst8)
                    b = lax.rem(j, 2)
                    soff = pl.multiple_of(j * 128, 8)
                    pltpu.make_async_copy(
                        tab.at[src_v.at[pl.ds(soff, 128)]],
                        rows_v.at[b], sem.at[b]).wait()
                    pltpu.sync_copy(rows_v.at[b],
                                    acc.at[dst8.at[lax.rem(j, 8)]], add=True)
                    return carry
                lax.fori_loop(0, CPT, body, 0)

            @pl.when(c == 0)
            def _():
                run(tab0)

            @pl.when(c == 1)
            def _():
                run(tab1)

            plsc.subcore_barrier()
            sliced(lambda r, n: pltpu.sync_copy(
                acc.at[pl.ds(r, n)], out.at[c, pl.ds(h * NH + r, n)]))
            plsc.subcore_barrier()

        phase(tlo, thi, stt, dtt0, out_tt, 0)
        phase(tlo, thi, stt, dtt1, out_tt, 1)
        phase(rlo, rhi, srr, drr0, out_rr, 0)
        phase(rlo, rhi, srr, drr1, out_rr, 1)
        phase(rlo, rhi, srt, drt0, out_rt, 0)
        phase(rlo, rhi, srt, drt1, out_rt, 1)

    return k


@functools.cache
def _sc_counts_kernel():
    mesh = plsc.VectorSubcoreMesh(core_axis_name="c", subcore_axis_name="s")
    csd = jax.ShapeDtypeStruct((NS, F), jnp.float32)

    @functools.partial(
        pl.kernel,
        out_type=(csd, csd, csd),
        mesh=mesh,
        scratch_types=[
            pltpu.VMEM((8, 128), jnp.int32),
            pltpu.VMEM((128, F), jnp.float32),
            pltpu.VMEM_SHARED((NH, F), jnp.float32),
        ],
    )
    def k(dtt0, dtt1, drr0, drr1, drt0, drt1, ones, zz,
          out_tt, out_rr, out_rt, dst8, ones_v, acc):
        c = lax.axis_index("c")
        s = lax.axis_index("s")
        r0 = s * RPTH

        def sliced(fn):
            @pl.when(s < 15)
            def _():
                fn(r0, RPTH)

            @pl.when(s == 15)
            def _():
                fn(15 * RPTH, NH - 15 * RPTH)

        pltpu.sync_copy(ones, ones_v)

        def cphase(dstm, out, h):
            sliced(lambda r, n: pltpu.sync_copy(zz.at[pl.ds(0, n)],
                                                acc.at[pl.ds(r, n)]))
            plsc.subcore_barrier()

            def body(j, carry):
                @pl.when(lax.rem(j, 8) == 0)
                def _():
                    pltpu.sync_copy(
                        dstm.at[s * (CPT // 8) + lax.div(j, 8)], dst8)
                pltpu.sync_copy(ones_v, acc.at[dst8.at[lax.rem(j, 8)]],
                                add=True)
                return carry
            lax.fori_loop(0, CPT, body, 0)
            plsc.subcore_barrier()
            sliced(lambda r, n: pltpu.sync_copy(
                acc.at[pl.ds(r, n)], out.at[pl.ds(h * NH + r, n)]))
            plsc.subcore_barrier()

        @pl.when(c == 0)
        def _():
            cphase(dtt0, out_tt, 0)
            cphase(dtt1, out_tt, 1)
            cphase(drt0, out_rt, 0)
            cphase(drt1, out_rt, 1)

        @pl.when(c == 1)
        def _():
            cphase(drr0, out_rr, 0)
            cphase(drr1, out_rr, 1)

    return k


def _tc_post(x, W1, Wp, g, b):
    nrows, kdim = x.shape
    R = 1000

    def body(x_ref, w1_ref, wp_ref, g_ref, b_ref, lo_ref, hi_ref):
        h = jnp.dot(x_ref[...], w1_ref[...], preferred_element_type=jnp.float32)
        h = _leaky(h)
        h = jnp.dot(h, wp_ref[...], preferred_element_type=jnp.float32)
        m = jnp.mean(h, axis=1, keepdims=True)
        v = jnp.mean((h - m) * (h - m), axis=1, keepdims=True)
        h = (h - m) * lax.rsqrt(v + 1e-5) * g_ref[...] + b_ref[...]
        h = _leaky(h)
        lo_ref[...] = h[:, :F]
        hi_ref[...] = h[:, F:]

    return pl.pallas_call(
        body,
        grid=(nrows // R,),
        in_specs=[
            pl.BlockSpec((R, kdim), lambda i: (i, 0)),
            pl.BlockSpec((kdim, HID), lambda i: (0, 0)),
            pl.BlockSpec((HID, HID), lambda i: (0, 0)),
            pl.BlockSpec((1, HID), lambda i: (0, 0)),
            pl.BlockSpec((1, HID), lambda i: (0, 0)),
        ],
        out_specs=[
            pl.BlockSpec((R, F), lambda i: (i, 0)),
            pl.BlockSpec((R, F), lambda i: (i, 0)),
        ],
        out_shape=[
            jax.ShapeDtypeStruct((nrows, F), jnp.float32),
            jax.ShapeDtypeStruct((nrows, F), jnp.float32),
        ],
    )(x, W1, Wp, g, b)


def _tc_combine_tgt(s_tt, s_rt, cnt_tt, cnt_rt, h_lo, h_hi,
                    wl_tt, wr_tt, b_tt, wl_rt, wr_rt, b_rt, final):
    R = 1000

    def body(stt_ref, srt_ref, ctt_ref, crt_ref, hlo_ref, hhi_ref,
             wltt_ref, wrtt_ref, btt_ref, wlrt_ref, wrrt_ref, brt_ref, *outs):
        ctt = jnp.maximum(ctt_ref[...], 1.0)
        crt = jnp.maximum(crt_ref[...], 1.0)
        y = jnp.dot(stt_ref[0] / ctt, wltt_ref[0:F, :],
                    preferred_element_type=jnp.float32)
        y = y + jnp.dot(stt_ref[1] / ctt, wltt_ref[F:, :],
                        preferred_element_type=jnp.float32)
        y = y + jnp.dot(srt_ref[0] / crt, wlrt_ref[0:F, :],
                        preferred_element_type=jnp.float32)
        y = y + jnp.dot(srt_ref[1] / crt, wlrt_ref[F:, :],
                        preferred_element_type=jnp.float32)
        y = y + jnp.dot(hlo_ref[...], wrtt_ref[0:F, :] + wrrt_ref[0:F, :],
                        preferred_element_type=jnp.float32)
        y = y + jnp.dot(hhi_ref[...], wrtt_ref[F:, :] + wrrt_ref[F:, :],
                        preferred_element_type=jnp.float32)
        y = y + btt_ref[...] + brt_ref[...]
        h = _leaky(0.5 * y)
        if final:
            outs[0][...] = h
        else:
            outs[0][...] = h[:, :F]
            outs[1][...] = h[:, F:]

    if final:
        out_specs = [pl.BlockSpec((R, HID), lambda i: (i, 0))]
        out_shape = [jax.ShapeDtypeStruct((N, HID), jnp.float32)]
    else:
        out_specs = [pl.BlockSpec((R, F), lambda i: (i, 0)),
                     pl.BlockSpec((R, F), lambda i: (i, 0))]
        out_shape = [jax.ShapeDtypeStruct((N, F), jnp.float32),
                     jax.ShapeDtypeStruct((N, F), jnp.float32)]

    smap = lambda i: (0, jnp.where(i < NB // R, i, i + 1), 0)
    smap2 = lambda i: (jnp.where(i < NB // R, i, i + 1), 0)
    return pl.pallas_call(
        body,
        grid=(N // R,),
        in_specs=[
            pl.BlockSpec((2, R, F), smap),
            pl.BlockSpec((2, R, F), smap),
            pl.BlockSpec((R, F), smap2),
            pl.BlockSpec((R, F), smap2),
            pl.BlockSpec((R, F), lambda i: (i, 0)),
            pl.BlockSpec((R, F), lambda i: (i, 0)),
            pl.BlockSpec((HID, HID), lambda i: (0, 0)),
            pl.BlockSpec((HID, HID), lambda i: (0, 0)),
            pl.BlockSpec((1, HID), lambda i: (0, 0)),
            pl.BlockSpec((HID, HID), lambda i: (0, 0)),
            pl.BlockSpec((HID, HID), lambda i: (0, 0)),
            pl.BlockSpec((1, HID), lambda i: (0, 0)),
        ],
        out_specs=out_specs,
        out_shape=out_shape,
    )(s_tt, s_rt, cnt_tt, cnt_rt, h_lo, h_hi,
      wl_tt, wr_tt, b_tt, wl_rt, wr_rt, b_rt)


def _tc_combine_ref(s_rr, cnt, h_lo, h_hi, wl_rr, wr_rr, b_rr, final):
    R = 1000

    def body(srr_ref, crr_ref, hlo_ref, hhi_ref,
             wl_ref, wr_ref, b_ref, *outs):
        crr = jnp.maximum(crr_ref[...], 1.0)
        y = jnp.dot(srr_ref[0] / crr, wl_ref[0:F, :],
                    preferred_element_type=jnp.float32)
        y = y + jnp.dot(srr_ref[1] / crr, wl_ref[F:, :],
                        preferred_element_type=jnp.float32)
        y = y + jnp.dot(hlo_ref[...], wr_ref[0:F, :],
                        preferred_element_type=jnp.float32)
        y = y + jnp.dot(hhi_ref[...], wr_ref[F:, :],
                        preferred_element_type=jnp.float32)
        y = y + b_ref[...]
        h = _leaky(y)
        if final:
            outs[0][...] = h
        else:
            outs[0][...] = h[:, :F]
            outs[1][...] = h[:, F:]

    if final:
        out_specs = [pl.BlockSpec((R, HID), lambda i: (i, 0))]
        out_shape = [jax.ShapeDtypeStruct((N, HID), jnp.float32)]
    else:
        out_specs = [pl.BlockSpec((R, F), lambda i: (i, 0)),
                     pl.BlockSpec((R, F), lambda i: (i, 0))]
        out_shape = [jax.ShapeDtypeStruct((N, F), jnp.float32),
                     jax.ShapeDtypeStruct((N, F), jnp.float32)]

    smap = lambda i: (0, jnp.where(i < NB // R, i, i + 1), 0)
    smap2 = lambda i: (jnp.where(i < NB // R, i, i + 1), 0)
    return pl.pallas_call(
        body,
        grid=(N // R,),
        in_specs=[
            pl.BlockSpec((2, R, F), smap),
            pl.BlockSpec((R, F), smap2),
            pl.BlockSpec((R, F), lambda i: (i, 0)),
            pl.BlockSpec((R, F), lambda i: (i, 0)),
            pl.BlockSpec((HID, HID), lambda i: (0, 0)),
            pl.BlockSpec((HID, HID), lambda i: (0, 0)),
            pl.BlockSpec((1, HID), lambda i: (0, 0)),
        ],
        out_specs=out_specs,
        out_shape=out_shape,
    )(s_rr, cnt, h_lo, h_hi, wl_rr, wr_rr, b_rr)


def _prep_edges(ei):
    e = ei.shape[1]
    pad = ECH * 128 - e
    spread = jnp.arange(pad, dtype=jnp.int32)
    src = jnp.concatenate([ei[0], spread % 4096])
    dst = jnp.concatenate([ei[1], N + (spread & 511)])
    trash = NB + (dst & 511)
    dst_h0 = jnp.where(dst < NB, dst, trash)
    dst_h1 = jnp.where(dst >= NB, dst - NB, trash)
    dst_h1 = jnp.where(dst_h1 >= NB, trash, dst_h1)
    dst_c = jnp.where(dst < N, dst, N + (dst & 127))
    g3 = (ECH // 8, 8, 128)
    return (src, dst_h0.reshape(g3), dst_h1.reshape(g3), dst_c.reshape(g3))


def kernel(x_target, x_reference, edge_index_tt, edge_index_rr, edge_index_rt,
           params):
    p = params
    src_tt, dtt0, dtt1, _ = _prep_edges(edge_index_tt)
    src_rr, drr0, drr1, _ = _prep_edges(edge_index_rr)
    src_rt, drt0, drt1, _ = _prep_edges(edge_index_rt)

    zeros_f = jnp.zeros((RPTH, F), jnp.float32)
    ones_r = jnp.ones((128, F), jnp.float32)

    g2 = p['ln_g'].reshape(1, HID)
    b2 = p['ln_b'].reshape(1, HID)

    ht_lo, ht_hi = _tc_post(x_target, p['W_win'], p['W_post'], g2, b2)
    hr_lo, hr_hi = _tc_post(x_reference, p['W_exp'], p['W_post'], g2, b2)

    cnt_tt, cnt_rr, cnt_rt = _sc_counts_kernel()(
        dtt0, dtt1, drr0, drr1, drt0, drt1, ones_r, zeros_f)

    names = ('Wl_tt', 'Wr_tt', 'b_tt', 'Wl_rr', 'Wr_rr', 'b_rr',
             'Wl_rt', 'Wr_rt', 'b_rt')
    ws = {n: jnp.stack([layer[n] for layer in p['layers']]) for n in names}

    def step(carry, w):
        ht_lo, ht_hi, hr_lo, hr_hi = carry
        s_tt, s_rr, s_rt = _sc_scatter3_kernel()(
            ht_lo, ht_hi, hr_lo, hr_hi, src_tt, dtt0, dtt1,
            src_rr, drr0, drr1, src_rt, drt0, drt1, zeros_f)
        nt_lo, nt_hi = _tc_combine_tgt(
            s_tt, s_rt, cnt_tt, cnt_rt, ht_lo, ht_hi,
            w['Wl_tt'], w['Wr_tt'], w['b_tt'].reshape(1, HID),
            w['Wl_rt'], w['Wr_rt'], w['b_rt'].reshape(1, HID), False)
        nr_lo, nr_hi = _tc_combine_ref(
            s_rr, cnt_rr, hr_lo, hr_hi,
            w['Wl_rr'], w['Wr_rr'], w['b_rr'].reshape(1, HID), False)
        return (nt_lo, nt_hi, nr_lo, nr_hi), None

    (ht_lo, ht_hi, hr_lo, hr_hi), _ = lax.scan(
        step, (ht_lo, ht_hi, hr_lo, hr_hi), ws)

    h_tgt = jnp.concatenate([ht_lo, ht_hi], axis=1)
    h_ref = jnp.concatenate([hr_lo, hr_hi], axis=1)
    return (h_tgt, h_ref)


# speedup vs baseline: 3.5738x; 1.0853x over previous
import functools

import jax
import jax.numpy as jnp
from jax import lax
from jax.experimental import pallas as pl
from jax.experimental.pallas import tpu as pltpu
from jax.experimental.pallas import tpu_sc as plsc

N = 10000
NPAD = 10240
RPT = NPAD // 16
NB = 5000
NH = 6000
RPTH = 376
NS = 2 * NH
ECH = 1280
CPT = ECH // 16
F = 128
CW = 16
HID = 256


def _leaky(x):
    return jnp.where(x >= 0.0, x, 0.2 * x)


@functools.cache
def _sc_scatter3_kernel():
    mesh = plsc.VectorSubcoreMesh(core_axis_name="c", subcore_axis_name="s")
    ssd = jax.ShapeDtypeStruct((2, NS, F), jnp.float32)
    ECT = CPT * 128

    @functools.partial(
        pl.kernel,
        out_type=(ssd, ssd, ssd),
        mesh=mesh,
        scratch_types=[
            pltpu.VMEM((ECT,), jnp.int32),
            pltpu.VMEM((8, 128), jnp.int32),
            pltpu.VMEM((3, 128, F), jnp.float32),
            pltpu.VMEM_SHARED((NH, F), jnp.float32),
            pltpu.SemaphoreType.DMA((3,)),
            pltpu.SemaphoreType.DMA((3,)),
        ],
    )
    def k(tlo, thi, rlo, rhi, stt, dtt0, dtt1, srr, drr0, drr1,
          srt, drt0, drt1, zz, out_tt, out_rr, out_rt,
          src_v, dst8, rows_v, acc, sem_g, sem_s):
        c = lax.axis_index("c")
        s = lax.axis_index("s")
        r0 = s * RPTH

        def sliced(fn):
            @pl.when(s < 15)
            def _():
                fn(r0, RPTH)

            @pl.when(s == 15)
            def _():
                fn(15 * RPTH, NH - 15 * RPTH)

        def phase(tab0, tab1, srcm, dstm, out, h):
            sliced(lambda r, n: pltpu.sync_copy(zz.at[pl.ds(0, n)],
                                                acc.at[pl.ds(r, n)]))
            pltpu.sync_copy(srcm.at[pl.ds(s * ECT, ECT)], src_v)
            plsc.subcore_barrier()

            def run(tab):
                def gstart(j):
                    soff = pl.multiple_of(j * 128, 8)
                    b = lax.rem(j, 3)
                    pltpu.async_copy(tab.at[src_v.at[pl.ds(soff, 128)]],
                                     rows_v.at[b], sem_g.at[b])

                def gwait(j):
                    soff = pl.multiple_of(j * 128, 8)
                    b = lax.rem(j, 3)
                    pltpu.make_async_copy(
                        tab.at[src_v.at[pl.ds(soff, 128)]],
                        rows_v.at[b], sem_g.at[b]).wait()

                def sstart(j):
                    b = lax.rem(j, 3)
                    pltpu.async_copy(rows_v.at[b],
                                     acc.at[dst8.at[lax.rem(j, 8)]],
                                     sem_s.at[b], add=True)

                def swait(j):
                    b = lax.rem(j, 3)
                    pltpu.make_async_copy(rows_v.at[b],
                                          acc.at[dst8.at[lax.rem(j, 8)]],
                                          sem_s.at[b]).wait()

                gstart(0)
                gstart(1)

                def body(j, carry):
                    @pl.when(j >= 1)
                    def _():
                        swait(j - 1)

                    @pl.when(lax.rem(j, 8) == 0)
                    def _():
                        pltpu.sync_copy(
                            dstm.at[s * (CPT // 8) + lax.div(j, 8)], dst8)
                    gwait(j)
                    sstart(j)

                    @pl.when(j + 2 < CPT)
                    def _():
                        gstart(j + 2)
                    return carry
                lax.fori_loop(0, CPT, body, 0)
                swait(CPT - 1)

            @pl.when(c == 0)
            def _():
                run(tab0)

            @pl.when(c == 1)
            def _():
                run(tab1)

            plsc.subcore_barrier()
            sliced(lambda r, n: pltpu.sync_copy(
                acc.at[pl.ds(r, n)], out.at[c, pl.ds(h * NH + r, n)]))
            plsc.subcore_barrier()

        phase(tlo, thi, stt, dtt0, out_tt, 0)
        phase(tlo, thi, stt, dtt1, out_tt, 1)
        phase(rlo, rhi, srr, drr0, out_rr, 0)
        phase(rlo, rhi, srr, drr1, out_rr, 1)
        phase(rlo, rhi, srt, drt0, out_rt, 0)
        phase(rlo, rhi, srt, drt1, out_rt, 1)

    return k


@functools.cache
def _sc_counts_kernel():
    mesh = plsc.VectorSubcoreMesh(core_axis_name="c", subcore_axis_name="s")
    csd = jax.ShapeDtypeStruct((NS, F), jnp.float32)

    @functools.partial(
        pl.kernel,
        out_type=(csd, csd, csd),
        mesh=mesh,
        scratch_types=[
            pltpu.VMEM((8, 128), jnp.int32),
            pltpu.VMEM((128, F), jnp.float32),
            pltpu.VMEM_SHARED((NH, F), jnp.float32),
        ],
    )
    def k(dtt0, dtt1, drr0, drr1, drt0, drt1, ones, zz,
          out_tt, out_rr, out_rt, dst8, ones_v, acc):
        c = lax.axis_index("c")
        s = lax.axis_index("s")
        r0 = s * RPTH

        def sliced(fn):
            @pl.when(s < 15)
            def _():
                fn(r0, RPTH)

            @pl.when(s == 15)
            def _():
                fn(15 * RPTH, NH - 15 * RPTH)

        pltpu.sync_copy(ones, ones_v)

        def cphase(dstm, out, h):
            sliced(lambda r, n: pltpu.sync_copy(zz.at[pl.ds(0, n)],
                                                acc.at[pl.ds(r, n)]))
            plsc.subcore_barrier()

            def body(j, carry):
                @pl.when(lax.rem(j, 8) == 0)
                def _():
                    pltpu.sync_copy(
                        dstm.at[s * (CPT // 8) + lax.div(j, 8)], dst8)
                pltpu.sync_copy(ones_v, acc.at[dst8.at[lax.rem(j, 8)]],
                                add=True)
                return carry
            lax.fori_loop(0, CPT, body, 0)
            plsc.subcore_barrier()
            sliced(lambda r, n: pltpu.sync_copy(
                acc.at[pl.ds(r, n)], out.at[pl.ds(h * NH + r, n)]))
            plsc.subcore_barrier()

        @pl.when(c == 0)
        def _():
            cphase(dtt0, out_tt, 0)
            cphase(dtt1, out_tt, 1)
            cphase(drt0, out_rt, 0)
            cphase(drt1, out_rt, 1)

        @pl.when(c == 1)
        def _():
            cphase(drr0, out_rr, 0)
            cphase(drr1, out_rr, 1)

    return k


def _tc_post(x, W1, Wp, g, b):
    nrows, kdim = x.shape
    R = 1000

    def body(x_ref, w1_ref, wp_ref, g_ref, b_ref, lo_ref, hi_ref):
        h = jnp.dot(x_ref[...], w1_ref[...], preferred_element_type=jnp.float32)
        h = _leaky(h)
        h = jnp.dot(h, wp_ref[...], preferred_element_type=jnp.float32)
        m = jnp.mean(h, axis=1, keepdims=True)
        v = jnp.mean((h - m) * (h - m), axis=1, keepdims=True)
        h = (h - m) * lax.rsqrt(v + 1e-5) * g_ref[...] + b_ref[...]
        h = _leaky(h)
        lo_ref[...] = h[:, :F]
        hi_ref[...] = h[:, F:]

    return pl.pallas_call(
        body,
        grid=(nrows // R,),
        in_specs=[
            pl.BlockSpec((R, kdim), lambda i: (i, 0)),
            pl.BlockSpec((kdim, HID), lambda i: (0, 0)),
            pl.BlockSpec((HID, HID), lambda i: (0, 0)),
            pl.BlockSpec((1, HID), lambda i: (0, 0)),
            pl.BlockSpec((1, HID), lambda i: (0, 0)),
        ],
        out_specs=[
            pl.BlockSpec((R, F), lambda i: (i, 0)),
            pl.BlockSpec((R, F), lambda i: (i, 0)),
        ],
        out_shape=[
            jax.ShapeDtypeStruct((nrows, F), jnp.float32),
            jax.ShapeDtypeStruct((nrows, F), jnp.float32),
        ],
    )(x, W1, Wp, g, b)


def _tc_combine_tgt(s_tt, s_rt, cnt_tt, cnt_rt, h_lo, h_hi,
                    wl_tt, wr_tt, b_tt, wl_rt, wr_rt, b_rt, final):
    R = 1000

    def body(stt_ref, srt_ref, ctt_ref, crt_ref, hlo_ref, hhi_ref,
             wltt_ref, wrtt_ref, btt_ref, wlrt_ref, wrrt_ref, brt_ref, *outs):
        ctt = jnp.maximum(ctt_ref[...], 1.0)
        crt = jnp.maximum(crt_ref[...], 1.0)
        y = jnp.dot(stt_ref[0] / ctt, wltt_ref[0:F, :],
                    preferred_element_type=jnp.float32)
        y = y + jnp.dot(stt_ref[1] / ctt, wltt_ref[F:, :],
                        preferred_element_type=jnp.float32)
        y = y + jnp.dot(srt_ref[0] / crt, wlrt_ref[0:F, :],
                        preferred_element_type=jnp.float32)
        y = y + jnp.dot(srt_ref[1] / crt, wlrt_ref[F:, :],
                        preferred_element_type=jnp.float32)
        y = y + jnp.dot(hlo_ref[...], wrtt_ref[0:F, :] + wrrt_ref[0:F, :],
                        preferred_element_type=jnp.float32)
        y = y + jnp.dot(hhi_ref[...], wrtt_ref[F:, :] + wrrt_ref[F:, :],
                        preferred_element_type=jnp.float32)
        y = y + btt_ref[...] + brt_ref[...]
        h = _leaky(0.5 * y)
        if final:
            outs[0][...] = h
        else:
            outs[0][...] = h[:, :F]
            outs[1][...] = h[:, F:]

    if final:
        out_specs = [pl.BlockSpec((R, HID), lambda i: (i, 0))]
        out_shape = [jax.ShapeDtypeStruct((N, HID), jnp.float32)]
    else:
        out_specs = [pl.BlockSpec((R, F), lambda i: (i, 0)),
                     pl.BlockSpec((R, F), lambda i: (i, 0))]
        out_shape = [jax.ShapeDtypeStruct((N, F), jnp.float32),
                     jax.ShapeDtypeStruct((N, F), jnp.float32)]

    smap = lambda i: (0, jnp.where(i < NB // R, i, i + 1), 0)
    smap2 = lambda i: (jnp.where(i < NB // R, i, i + 1), 0)
    return pl.pallas_call(
        body,
        grid=(N // R,),
        in_specs=[
            pl.BlockSpec((2, R, F), smap),
            pl.BlockSpec((2, R, F), smap),
            pl.BlockSpec((R, F), smap2),
            pl.BlockSpec((R, F), smap2),
            pl.BlockSpec((R, F), lambda i: (i, 0)),
            pl.BlockSpec((R, F), lambda i: (i, 0)),
            pl.BlockSpec((HID, HID), lambda i: (0, 0)),
            pl.BlockSpec((HID, HID), lambda i: (0, 0)),
            pl.BlockSpec((1, HID), lambda i: (0, 0)),
            pl.BlockSpec((HID, HID), lambda i: (0, 0)),
            pl.BlockSpec((HID, HID), lambda i: (0, 0)),
            pl.BlockSpec((1, HID), lambda i: (0, 0)),
        ],
        out_specs=out_specs,
        out_shape=out_shape,
    )(s_tt, s_rt, cnt_tt, cnt_rt, h_lo, h_hi,
      wl_tt, wr_tt, b_tt, wl_rt, wr_rt, b_rt)


def _tc_combine_ref(s_rr, cnt, h_lo, h_hi, wl_rr, wr_rr, b_rr, final):
    R = 1000

    def body(srr_ref, crr_ref, hlo_ref, hhi_ref,
             wl_ref, wr_ref, b_ref, *outs):
        crr = jnp.maximum(crr_ref[...], 1.0)
        y = jnp.dot(srr_ref[0] / crr, wl_ref[0:F, :],
                    preferred_element_type=jnp.float32)
        y = y + jnp.dot(srr_ref[1] / crr, wl_ref[F:, :],
                        preferred_element_type=jnp.float32)
        y = y + jnp.dot(hlo_ref[...], wr_ref[0:F, :],
                        preferred_element_type=jnp.float32)
        y = y + jnp.dot(hhi_ref[...], wr_ref[F:, :],
                        preferred_element_type=jnp.float32)
        y = y + b_ref[...]
        h = _leaky(y)
        if final:
            outs[0][...] = h
        else:
            outs[0][...] = h[:, :F]
            outs[1][...] = h[:, F:]

    if final:
        out_specs = [pl.BlockSpec((R, HID), lambda i: (i, 0))]
        out_shape = [jax.ShapeDtypeStruct((N, HID), jnp.float32)]
    else:
        out_specs = [pl.BlockSpec((R, F), lambda i: (i, 0)),
                     pl.BlockSpec((R, F), lambda i: (i, 0))]
        out_shape = [jax.ShapeDtypeStruct((N, F), jnp.float32),
                     jax.ShapeDtypeStruct((N, F), jnp.float32)]

    smap = lambda i: (0, jnp.where(i < NB // R, i, i + 1), 0)
    smap2 = lambda i: (jnp.where(i < NB // R, i, i + 1), 0)
    return pl.pallas_call(
        body,
        grid=(N // R,),
        in_specs=[
            pl.BlockSpec((2, R, F), smap),
            pl.BlockSpec((R, F), smap2),
            pl.BlockSpec((R, F), lambda i: (i, 0)),
            pl.BlockSpec((R, F), lambda i: (i, 0)),
            pl.BlockSpec((HID, HID), lambda i: (0, 0)),
            pl.BlockSpec((HID, HID), lambda i: (0, 0)),
            pl.BlockSpec((1, HID), lambda i: (0, 0)),
        ],
        out_specs=out_specs,
        out_shape=out_shape,
    )(s_rr, cnt, h_lo, h_hi, wl_rr, wr_rr, b_rr)


def _prep_edges(ei):
    e = ei.shape[1]
    pad = ECH * 128 - e
    spread = jnp.arange(pad, dtype=jnp.int32)
    src = jnp.concatenate([ei[0], spread % 4096])
    dst = jnp.concatenate([ei[1], N + (spread & 511)])
    trash = NB + (dst & 511)
    dst_h0 = jnp.where(dst < NB, dst, trash)
    dst_h1 = jnp.where(dst >= NB, dst - NB, trash)
    dst_h1 = jnp.where(dst_h1 >= NB, trash, dst_h1)
    dst_c = jnp.where(dst < N, dst, N + (dst & 127))
    g3 = (ECH // 8, 8, 128)
    return (src, dst_h0.reshape(g3), dst_h1.reshape(g3), dst_c.reshape(g3))


def kernel(x_target, x_reference, edge_index_tt, edge_index_rr, edge_index_rt,
           params):
    p = params
    src_tt, dtt0, dtt1, _ = _prep_edges(edge_index_tt)
    src_rr, drr0, drr1, _ = _prep_edges(edge_index_rr)
    src_rt, drt0, drt1, _ = _prep_edges(edge_index_rt)

    zeros_f = jnp.zeros((RPTH, F), jnp.float32)
    ones_r = jnp.ones((128, F), jnp.float32)

    g2 = p['ln_g'].reshape(1, HID)
    b2 = p['ln_b'].reshape(1, HID)

    ht_lo, ht_hi = _tc_post(x_target, p['W_win'], p['W_post'], g2, b2)
    hr_lo, hr_hi = _tc_post(x_reference, p['W_exp'], p['W_post'], g2, b2)

    cnt_tt, cnt_rr, cnt_rt = _sc_counts_kernel()(
        dtt0, dtt1, drr0, drr1, drt0, drt1, ones_r, zeros_f)

    names = ('Wl_tt', 'Wr_tt', 'b_tt', 'Wl_rr', 'Wr_rr', 'b_rr',
             'Wl_rt', 'Wr_rt', 'b_rt')
    ws = {n: jnp.stack([layer[n] for layer in p['layers']]) for n in names}

    def step(carry, w):
        ht_lo, ht_hi, hr_lo, hr_hi = carry
        s_tt, s_rr, s_rt = _sc_scatter3_kernel()(
            ht_lo, ht_hi, hr_lo, hr_hi, src_tt, dtt0, dtt1,
            src_rr, drr0, drr1, src_rt, drt0, drt1, zeros_f)
        nt_lo, nt_hi = _tc_combine_tgt(
            s_tt, s_rt, cnt_tt, cnt_rt, ht_lo, ht_hi,
            w['Wl_tt'], w['Wr_tt'], w['b_tt'].reshape(1, HID),
            w['Wl_rt'], w['Wr_rt'], w['b_rt'].reshape(1, HID), False)
        nr_lo, nr_hi = _tc_combine_ref(
            s_rr, cnt_rr, hr_lo, hr_hi,
            w['Wl_rr'], w['Wr_rr'], w['b_rr'].reshape(1, HID), False)
        return (nt_lo, nt_hi, nr_lo, nr_hi), None

    (ht_lo, ht_hi, hr_lo, hr_hi), _ = lax.scan(
        step, (ht_lo, ht_hi, hr_lo, hr_hi), ws)

    h_tgt = jnp.concatenate([ht_lo, ht_hi], axis=1)
    h_ref = jnp.concatenate([hr_lo, hr_hi], axis=1)
    return (h_tgt, h_ref)


# speedup vs baseline: 4.9659x; 1.3895x over previous
import functools

import jax
import jax.numpy as jnp
from jax import lax
from jax.experimental import pallas as pl
from jax.experimental.pallas import tpu as pltpu
from jax.experimental.pallas import tpu_sc as plsc

N = 10000
NPAD = 10240
RPT = NPAD // 16
NB = 5000
NH = 6000
RPTH = 376
NS = 2 * NH
ECH = 1280
CPT = ECH // 16
F = 128
CW = 16
HID = 256


def _leaky(x):
    return jnp.where(x >= 0.0, x, 0.2 * x)


@functools.cache
def _sc_scatter3_kernel():
    mesh = plsc.VectorSubcoreMesh(core_axis_name="c", subcore_axis_name="s")
    ssd = jax.ShapeDtypeStruct((2, NS, F), jnp.float32)
    ECT = CPT * 128

    @functools.partial(
        pl.kernel,
        out_type=(ssd, ssd, ssd),
        mesh=mesh,
        compiler_params=pltpu.CompilerParams(needs_layout_passes=False),
        scratch_types=[
            pltpu.VMEM((ECT,), jnp.int32),
            pltpu.VMEM((8, 128), jnp.int32),
            pltpu.VMEM((3, 128, F), jnp.float32),
            pltpu.VMEM((256,), jnp.int32),
            pltpu.VMEM_SHARED((NH, F), jnp.float32),
            pltpu.SemaphoreType.DMA((3,)),
            pltpu.SemaphoreType.DMA((3,)),
        ],
    )
    def k(tlo, thi, rlo, rhi,
          stt0, dtt0, ntt0, stt1, dtt1, ntt1,
          srr0, drr0, nrr0, srr1, drr1, nrr1,
          srt0, drt0, nrt0, srt1, drt1, nrt1,
          zz, out_tt, out_rr, out_rt,
          src_v, dst8, rows_v, nc_v, acc, sem_g, sem_s):
        c = lax.axis_index("c")
        s = lax.axis_index("s")
        r0 = s * RPTH

        def sliced(fn):
            @pl.when(s < 15)
            def _():
                fn(r0, RPTH)

            @pl.when(s == 15)
            def _():
                fn(15 * RPTH, NH - 15 * RPTH)

        def phase(tab0, tab1, srcm, dstm, ncm, out, h):
            sliced(lambda r, n: pltpu.sync_copy(zz.at[pl.ds(0, n)],
                                                acc.at[pl.ds(r, n)]))
            pltpu.sync_copy(srcm.at[pl.ds(s * ECT, ECT)], src_v)
            pltpu.sync_copy(ncm, nc_v)
            ncvec = nc_v[pl.ds(pl.multiple_of(s * 16, 8), 16)]
            nc = jnp.max(ncvec)
            plsc.subcore_barrier()

            def run(tab):
                def gstart(j):
                    soff = pl.multiple_of(j * 128, 8)
                    b = lax.rem(j, 3)
                    pltpu.async_copy(tab.at[src_v.at[pl.ds(soff, 128)]],
                                     rows_v.at[b], sem_g.at[b])

                def gwait(j):
                    soff = pl.multiple_of(j * 128, 8)
                    b = lax.rem(j, 3)
                    pltpu.make_async_copy(
                        tab.at[src_v.at[pl.ds(soff, 128)]],
                        rows_v.at[b], sem_g.at[b]).wait()

                def sstart(j):
                    b = lax.rem(j, 3)
                    pltpu.async_copy(rows_v.at[b],
                                     acc.at[dst8.at[lax.rem(j, 8)]],
                                     sem_s.at[b], add=True)

                def swait(j):
                    b = lax.rem(j, 3)
                    pltpu.make_async_copy(rows_v.at[b],
                                          acc.at[dst8.at[lax.rem(j, 8)]],
                                          sem_s.at[b]).wait()

                @pl.when(nc >= 1)
                def _():
                    gstart(0)

                @pl.when(nc >= 2)
                def _():
                    gstart(1)

                def body(j, carry):
                    @pl.when(j >= 1)
                    def _():
                        swait(j - 1)

                    @pl.when(lax.rem(j, 8) == 0)
                    def _():
                        pltpu.sync_copy(
                            dstm.at[s * (CPT // 8) + lax.div(j, 8)], dst8)
                    gwait(j)
                    sstart(j)

                    @pl.when(j + 2 < nc)
                    def _():
                        gstart(j + 2)
                    return carry
                lax.fori_loop(0, nc, body, 0)

                @pl.when(nc >= 1)
                def _():
                    swait(nc - 1)

            @pl.when(c == 0)
            def _():
                run(tab0)

            @pl.when(c == 1)
            def _():
                run(tab1)

            plsc.subcore_barrier()
            sliced(lambda r, n: pltpu.sync_copy(
                acc.at[pl.ds(r, n)], out.at[c, pl.ds(h * NH + r, n)]))
            plsc.subcore_barrier()

        phase(tlo, thi, stt0, dtt0, ntt0, out_tt, 0)
        phase(tlo, thi, stt1, dtt1, ntt1, out_tt, 1)
        phase(rlo, rhi, srr0, drr0, nrr0, out_rr, 0)
        phase(rlo, rhi, srr1, drr1, nrr1, out_rr, 1)
        phase(rlo, rhi, srt0, drt0, nrt0, out_rt, 0)
        phase(rlo, rhi, srt1, drt1, nrt1, out_rt, 1)

    return k


@functools.cache
def _sc_partition_kernel():
    mesh = plsc.VectorSubcoreMesh(core_axis_name="c", subcore_axis_name="s")
    ECT = CPT * 128
    lsd = jax.ShapeDtypeStruct((16 * ECT,), jnp.int32)
    ncd = jax.ShapeDtypeStruct((256,), jnp.int32)

    @functools.partial(
        pl.kernel,
        out_type=tuple([lsd, lsd, ncd] * 6),
        mesh=mesh,
        compiler_params=pltpu.CompilerParams(needs_layout_passes=False),
        scratch_types=[
            pltpu.VMEM((ECT,), jnp.int32),
            pltpu.VMEM((ECT,), jnp.int32),
            pltpu.VMEM((ECT,), jnp.int32),
            pltpu.VMEM((ECT,), jnp.int32),
            pltpu.VMEM((ECT,), jnp.int32),
            pltpu.VMEM((ECT,), jnp.int32),
            pltpu.VMEM((16,), jnp.int32),
        ],
    )
    def k(stt, dtt, srr, drr, srt, drt,
          ps_tt0, pd_tt0, nc_tt0, ps_tt1, pd_tt1, nc_tt1,
          ps_rr0, pd_rr0, nc_rr0, ps_rr1, pd_rr1, nc_rr1,
          ps_rt0, pd_rt0, nc_rt0, ps_rt1, pd_rt1, nc_rt1,
          src_v, dst_v, ls0, ld0, ls1, ld1, ncb):
        c = lax.axis_index("c")
        s = lax.axis_index("s")
        lane = lax.iota(jnp.int32, 16)

        def phase(srcm, dstm, outs0, outs1):
            pltpu.sync_copy(srcm.at[pl.ds(s * ECT, ECT)], src_v)
            pltpu.sync_copy(dstm.at[pl.ds(s * ECT, ECT)], dst_v)

            def pre(i, carry):
                off = pl.multiple_of(i * 16, 8)
                iv = lane + i * 16
                ls0[pl.ds(off, 16)] = iv & 4095
                ld0[pl.ds(off, 16)] = NB + (iv & 511)
                ls1[pl.ds(off, 16)] = iv & 4095
                ld1[pl.ds(off, 16)] = NB + (iv & 511)
                return carry
            lax.fori_loop(0, ECT // 16, pre, 0)

            def body(g, carry):
                o0, o1 = carry
                off = pl.multiple_of(g * 16, 8)
                sv = src_v[pl.ds(off, 16)]
                dv = dst_v[pl.ds(off, 16)]
                m0 = dv < NB
                pos0 = o0 + plsc.cumsum(jnp.where(m0, 1, 0)) - 1
                plsc.store_scatter(ls0, [pos0], sv, mask=m0)
                plsc.store_scatter(ld0, [pos0], dv, mask=m0)
                o0 = o0 + plsc.all_reduce_population_count(m0)
                m1 = jnp.logical_and(dv >= NB, dv < N)
                pos1 = o1 + plsc.cumsum(jnp.where(m1, 1, 0)) - 1
                plsc.store_scatter(ls1, [pos1], sv, mask=m1)
                plsc.store_scatter(ld1, [pos1], dv - NB, mask=m1)
                o1 = o1 + plsc.all_reduce_population_count(m1)
                return (o0, o1)
            zero = jnp.zeros((16,), jnp.int32)
            o0, o1 = lax.fori_loop(0, ECT // 16, body, (zero, zero))

            for (ps, pd, nc), ls, ld, ov in ((outs0, ls0, ld0, o0),
                                             (outs1, ls1, ld1, o1)):
                pltpu.sync_copy(ls, ps.at[pl.ds(s * ECT, ECT)])
                pltpu.sync_copy(ld, pd.at[pl.ds(s * ECT, ECT)])
                ncb[...] = lax.shift_right_logical(ov + 127, 7)
                pltpu.sync_copy(ncb, nc.at[pl.ds(s * 16, 16)])

        @pl.when(c == 0)
        def _():
            phase(stt, dtt, (ps_tt0, pd_tt0, nc_tt0), (ps_tt1, pd_tt1, nc_tt1))
            phase(srt, drt, (ps_rt0, pd_rt0, nc_rt0), (ps_rt1, pd_rt1, nc_rt1))

        @pl.when(c == 1)
        def _():
            phase(srr, drr, (ps_rr0, pd_rr0, nc_rr0), (ps_rr1, pd_rr1, nc_rr1))

    return k


@functools.cache
def _sc_counts_kernel():
    mesh = plsc.VectorSubcoreMesh(core_axis_name="c", subcore_axis_name="s")
    csd = jax.ShapeDtypeStruct((NS, F), jnp.float32)

    @functools.partial(
        pl.kernel,
        out_type=(csd, csd, csd),
        mesh=mesh,
        scratch_types=[
            pltpu.VMEM((8, 128), jnp.int32),
            pltpu.VMEM((128, F), jnp.float32),
            pltpu.VMEM_SHARED((NH, F), jnp.float32),
        ],
    )
    def k(dtt0, dtt1, drr0, drr1, drt0, drt1, ones, zz,
          out_tt, out_rr, out_rt, dst8, ones_v, acc):
        c = lax.axis_index("c")
        s = lax.axis_index("s")
        r0 = s * RPTH

        def sliced(fn):
            @pl.when(s < 15)
            def _():
                fn(r0, RPTH)

            @pl.when(s == 15)
            def _():
                fn(15 * RPTH, NH - 15 * RPTH)

        pltpu.sync_copy(ones, ones_v)

        def cphase(dstm, out, h):
            sliced(lambda r, n: pltpu.sync_copy(zz.at[pl.ds(0, n)],
                                                acc.at[pl.ds(r, n)]))
            plsc.subcore_barrier()

            def body(j, carry):
                @pl.when(lax.rem(j, 8) == 0)
                def _():
                    pltpu.sync_copy(
                        dstm.at[s * (CPT // 8) + lax.div(j, 8)], dst8)
                pltpu.sync_copy(ones_v, acc.at[dst8.at[lax.rem(j, 8)]],
                                add=True)
                return carry
            lax.fori_loop(0, CPT, body, 0)
            plsc.subcore_barrier()
            sliced(lambda r, n: pltpu.sync_copy(
                acc.at[pl.ds(r, n)], out.at[pl.ds(h * NH + r, n)]))
            plsc.subcore_barrier()

        @pl.when(c == 0)
        def _():
            cphase(dtt0, out_tt, 0)
            cphase(dtt1, out_tt, 1)
            cphase(drt0, out_rt, 0)
            cphase(drt1, out_rt, 1)

        @pl.when(c == 1)
        def _():
            cphase(drr0, out_rr, 0)
            cphase(drr1, out_rr, 1)

    return k


def _tc_post(x, W1, Wp, g, b):
    nrows, kdim = x.shape
    R = 1000

    def body(x_ref, w1_ref, wp_ref, g_ref, b_ref, lo_ref, hi_ref):
        h = jnp.dot(x_ref[...], w1_ref[...], preferred_element_type=jnp.float32)
        h = _leaky(h)
        h = jnp.dot(h, wp_ref[...], preferred_element_type=jnp.float32)
        m = jnp.mean(h, axis=1, keepdims=True)
        v = jnp.mean((h - m) * (h - m), axis=1, keepdims=True)
        h = (h - m) * lax.rsqrt(v + 1e-5) * g_ref[...] + b_ref[...]
        h = _leaky(h)
        lo_ref[...] = h[:, :F]
        hi_ref[...] = h[:, F:]

    return pl.pallas_call(
        body,
        grid=(nrows // R,),
        in_specs=[
            pl.BlockSpec((R, kdim), lambda i: (i, 0)),
            pl.BlockSpec((kdim, HID), lambda i: (0, 0)),
            pl.BlockSpec((HID, HID), lambda i: (0, 0)),
            pl.BlockSpec((1, HID), lambda i: (0, 0)),
            pl.BlockSpec((1, HID), lambda i: (0, 0)),
        ],
        out_specs=[
            pl.BlockSpec((R, F), lambda i: (i, 0)),
            pl.BlockSpec((R, F), lambda i: (i, 0)),
        ],
        out_shape=[
            jax.ShapeDtypeStruct((nrows, F), jnp.float32),
            jax.ShapeDtypeStruct((nrows, F), jnp.float32),
        ],
    )(x, W1, Wp, g, b)


def _tc_combine_tgt(s_tt, s_rt, cnt_tt, cnt_rt, h_lo, h_hi,
                    wl_tt, wr_tt, b_tt, wl_rt, wr_rt, b_rt, final):
    R = 1000

    def body(stt_ref, srt_ref, ctt_ref, crt_ref, hlo_ref, hhi_ref,
             wltt_ref, wrtt_ref, btt_ref, wlrt_ref, wrrt_ref, brt_ref, *outs):
        ctt = jnp.maximum(ctt_ref[...], 1.0)
        crt = jnp.maximum(crt_ref[...], 1.0)
        y = jnp.dot(stt_ref[0] / ctt, wltt_ref[0:F, :],
                    preferred_element_type=jnp.float32)
        y = y + jnp.dot(stt_ref[1] / ctt, wltt_ref[F:, :],
                        preferred_element_type=jnp.float32)
        y = y + jnp.dot(srt_ref[0] / crt, wlrt_ref[0:F, :],
                        preferred_element_type=jnp.float32)
        y = y + jnp.dot(srt_ref[1] / crt, wlrt_ref[F:, :],
                        preferred_element_type=jnp.float32)
        y = y + jnp.dot(hlo_ref[...], wrtt_ref[0:F, :] + wrrt_ref[0:F, :],
                        preferred_element_type=jnp.float32)
        y = y + jnp.dot(hhi_ref[...], wrtt_ref[F:, :] + wrrt_ref[F:, :],
                        preferred_element_type=jnp.float32)
        y = y + btt_ref[...] + brt_ref[...]
        h = _leaky(0.5 * y)
        if final:
            outs[0][...] = h
        else:
            outs[0][...] = h[:, :F]
            outs[1][...] = h[:, F:]

    if final:
        out_specs = [pl.BlockSpec((R, HID), lambda i: (i, 0))]
        out_shape = [jax.ShapeDtypeStruct((N, HID), jnp.float32)]
    else:
        out_specs = [pl.BlockSpec((R, F), lambda i: (i, 0)),
                     pl.BlockSpec((R, F), lambda i: (i, 0))]
        out_shape = [jax.ShapeDtypeStruct((N, F), jnp.float32),
                     jax.ShapeDtypeStruct((N, F), jnp.float32)]

    smap = lambda i: (0, jnp.where(i < NB // R, i, i + 1), 0)
    smap2 = lambda i: (jnp.where(i < NB // R, i, i + 1), 0)
    return pl.pallas_call(
        body,
        grid=(N // R,),
        in_specs=[
            pl.BlockSpec((2, R, F), smap),
            pl.BlockSpec((2, R, F), smap),
            pl.BlockSpec((R, F), smap2),
            pl.BlockSpec((R, F), smap2),
            pl.BlockSpec((R, F), lambda i: (i, 0)),
            pl.BlockSpec((R, F), lambda i: (i, 0)),
            pl.BlockSpec((HID, HID), lambda i: (0, 0)),
            pl.BlockSpec((HID, HID), lambda i: (0, 0)),
            pl.BlockSpec((1, HID), lambda i: (0, 0)),
            pl.BlockSpec((HID, HID), lambda i: (0, 0)),
            pl.BlockSpec((HID, HID), lambda i: (0, 0)),
            pl.BlockSpec((1, HID), lambda i: (0, 0)),
        ],
        out_specs=out_specs,
        out_shape=out_shape,
    )(s_tt, s_rt, cnt_tt, cnt_rt, h_lo, h_hi,
      wl_tt, wr_tt, b_tt, wl_rt, wr_rt, b_rt)


def _tc_combine_ref(s_rr, cnt, h_lo, h_hi, wl_rr, wr_rr, b_rr, final):
    R = 1000

    def body(srr_ref, crr_ref, hlo_ref, hhi_ref,
             wl_ref, wr_ref, b_ref, *outs):
        crr = jnp.maximum(crr_ref[...], 1.0)
        y = jnp.dot(srr_ref[0] / crr, wl_ref[0:F, :],
                    preferred_element_type=jnp.float32)
        y = y + jnp.dot(srr_ref[1] / crr, wl_ref[F:, :],
                        preferred_element_type=jnp.float32)
        y = y + jnp.dot(hlo_ref[...], wr_ref[0:F, :],
                        preferred_element_type=jnp.float32)
        y = y + jnp.dot(hhi_ref[...], wr_ref[F:, :],
                        preferred_element_type=jnp.float32)
        y = y + b_ref[...]
        h = _leaky(y)
        if final:
            outs[0][...] = h
        else:
            outs[0][...] = h[:, :F]
            outs[1][...] = h[:, F:]

    if final:
        out_specs = [pl.BlockSpec((R, HID), lambda i: (i, 0))]
        out_shape = [jax.ShapeDtypeStruct((N, HID), jnp.float32)]
    else:
        out_specs = [pl.BlockSpec((R, F), lambda i: (i, 0)),
                     pl.BlockSpec((R, F), lambda i: (i, 0))]
        out_shape = [jax.ShapeDtypeStruct((N, F), jnp.float32),
                     jax.ShapeDtypeStruct((N, F), jnp.float32)]

    smap = lambda i: (0, jnp.where(i < NB // R, i, i + 1), 0)
    smap2 = lambda i: (jnp.where(i < NB // R, i, i + 1), 0)
    return pl.pallas_call(
        body,
        grid=(N // R,),
        in_specs=[
            pl.BlockSpec((2, R, F), smap),
            pl.BlockSpec((R, F), smap2),
            pl.BlockSpec((R, F), lambda i: (i, 0)),
            pl.BlockSpec((R, F), lambda i: (i, 0)),
            pl.BlockSpec((HID, HID), lambda i: (0, 0)),
            pl.BlockSpec((HID, HID), lambda i: (0, 0)),
            pl.BlockSpec((1, HID), lambda i: (0, 0)),
        ],
        out_specs=out_specs,
        out_shape=out_shape,
    )(s_rr, cnt, h_lo, h_hi, wl_rr, wr_rr, b_rr)


def _prep_edges(ei):
    e = ei.shape[1]
    pad = ECH * 128 - e
    spread = jnp.arange(pad, dtype=jnp.int32)
    src = jnp.concatenate([ei[0], spread % 4096])
    dst = jnp.concatenate([ei[1], N + (spread & 511)])
    trash = NB + (dst & 511)
    dst_h0 = jnp.where(dst < NB, dst, trash)
    dst_h1 = jnp.where(dst >= NB, dst - NB, trash)
    dst_h1 = jnp.where(dst_h1 >= NB, trash, dst_h1)
    g3 = (ECH // 8, 8, 128)
    return (src, dst_h0.reshape(g3), dst_h1.reshape(g3), dst)


def kernel(x_target, x_reference, edge_index_tt, edge_index_rr, edge_index_rt,
           params):
    p = params
    src_tt, dtt0, dtt1, dfl_tt = _prep_edges(edge_index_tt)
    src_rr, drr0, drr1, dfl_rr = _prep_edges(edge_index_rr)
    src_rt, drt0, drt1, dfl_rt = _prep_edges(edge_index_rt)

    zeros_f = jnp.zeros((RPTH, F), jnp.float32)
    ones_r = jnp.ones((128, F), jnp.float32)

    g2 = p['ln_g'].reshape(1, HID)
    b2 = p['ln_b'].reshape(1, HID)

    ht_lo, ht_hi = _tc_post(x_target, p['W_win'], p['W_post'], g2, b2)
    hr_lo, hr_hi = _tc_post(x_reference, p['W_exp'], p['W_post'], g2, b2)

    cnt_tt, cnt_rr, cnt_rt = _sc_counts_kernel()(
        dtt0, dtt1, drr0, drr1, drt0, drt1, ones_r, zeros_f)

    (ps_tt0, pd_tt0, nc_tt0, ps_tt1, pd_tt1, nc_tt1,
     ps_rr0, pd_rr0, nc_rr0, ps_rr1, pd_rr1, nc_rr1,
     ps_rt0, pd_rt0, nc_rt0, ps_rt1, pd_rt1, nc_rt1) = _sc_partition_kernel()(
        src_tt, dfl_tt, src_rr, dfl_rr, src_rt, dfl_rt)
    rs = lambda a: a.reshape(ECH // 8, 8, 128)
    pd_tt0, pd_tt1 = rs(pd_tt0), rs(pd_tt1)
    pd_rr0, pd_rr1 = rs(pd_rr0), rs(pd_rr1)
    pd_rt0, pd_rt1 = rs(pd_rt0), rs(pd_rt1)

    names = ('Wl_tt', 'Wr_tt', 'b_tt', 'Wl_rr', 'Wr_rr', 'b_rr',
             'Wl_rt', 'Wr_rt', 'b_rt')
    ws = {n: jnp.stack([layer[n] for layer in p['layers']]) for n in names}

    def step(carry, w):
        ht_lo, ht_hi, hr_lo, hr_hi = carry
        s_tt, s_rr, s_rt = _sc_scatter3_kernel()(
            ht_lo, ht_hi, hr_lo, hr_hi,
            ps_tt0, pd_tt0, nc_tt0, ps_tt1, pd_tt1, nc_tt1,
            ps_rr0, pd_rr0, nc_rr0, ps_rr1, pd_rr1, nc_rr1,
            ps_rt0, pd_rt0, nc_rt0, ps_rt1, pd_rt1, nc_rt1, zeros_f)
        nt_lo, nt_hi = _tc_combine_tgt(
            s_tt, s_rt, cnt_tt, cnt_rt, ht_lo, ht_hi,
            w['Wl_tt'], w['Wr_tt'], w['b_tt'].reshape(1, HID),
            w['Wl_rt'], w['Wr_rt'], w['b_rt'].reshape(1, HID), False)
        nr_lo, nr_hi = _tc_combine_ref(
            s_rr, cnt_rr, hr_lo, hr_hi,
            w['Wl_rr'], w['Wr_rr'], w['b_rr'].reshape(1, HID), False)
        return (nt_lo, nt_hi, nr_lo, nr_hi), None

    (ht_lo, ht_hi, hr_lo, hr_hi), _ = lax.scan(
        step, (ht_lo, ht_hi, hr_lo, hr_hi), ws)

    h_tgt = jnp.concatenate([ht_lo, ht_hi], axis=1)
    h_ref = jnp.concatenate([hr_lo, hr_hi], axis=1)
    return (h_tgt, h_ref)


# speedup vs baseline: 5.4605x; 1.0996x over previous
import functools

import jax
import jax.numpy as jnp
from jax import lax
from jax.experimental import pallas as pl
from jax.experimental.pallas import tpu as pltpu
from jax.experimental.pallas import tpu_sc as plsc

N = 10000
NPAD = 10240
RPT = NPAD // 16
NB = 5000
NH = 6000
RPTH = 376
NS = 2 * NH
ECH = 1280
CPT = ECH // 16
F = 128
CW = 16
HID = 256


def _leaky(x):
    return jnp.where(x >= 0.0, x, 0.2 * x)


@functools.cache
def _sc_scatter3_kernel():
    mesh = plsc.VectorSubcoreMesh(core_axis_name="c", subcore_axis_name="s")
    ssd = jax.ShapeDtypeStruct((2, NS, F), jnp.float32)
    ECT = CPT * 128

    @functools.partial(
        pl.kernel,
        out_type=(ssd, ssd, ssd),
        mesh=mesh,
        compiler_params=pltpu.CompilerParams(needs_layout_passes=False),
        scratch_types=[
            pltpu.VMEM((ECT,), jnp.int32),
            pltpu.VMEM((8, 128), jnp.int32),
            pltpu.VMEM((3, 128, F), jnp.float32),
            pltpu.VMEM((256,), jnp.int32),
            pltpu.VMEM_SHARED((NH, F), jnp.float32),
            pltpu.SemaphoreType.DMA((3,)),
            pltpu.SemaphoreType.DMA((3,)),
        ],
    )
    def k(tlo, thi, rlo, rhi,
          stt0, dtt0, ntt0, stt1, dtt1, ntt1,
          srr0, drr0, nrr0, srr1, drr1, nrr1,
          srt0, drt0, nrt0, srt1, drt1, nrt1,
          zz, out_tt, out_rr, out_rt,
          src_v, dst8, rows_v, nc_v, acc, sem_g, sem_s):
        c = lax.axis_index("c")
        s = lax.axis_index("s")
        r0 = s * RPTH

        def sliced(fn):
            @pl.when(s < 15)
            def _():
                fn(r0, RPTH)

            @pl.when(s == 15)
            def _():
                fn(15 * RPTH, NH - 15 * RPTH)

        def phase(tab0, tab1, srcm, dstm, ncm, out, h):
            sliced(lambda r, n: pltpu.sync_copy(zz.at[pl.ds(0, n)],
                                                acc.at[pl.ds(r, n)]))
            pltpu.sync_copy(srcm.at[pl.ds(s * ECT, ECT)], src_v)
            pltpu.sync_copy(ncm, nc_v)
            ncvec = nc_v[pl.ds(pl.multiple_of(s * 16, 8), 16)]
            nc = jnp.max(ncvec)
            plsc.subcore_barrier()

            def run(tab):
                def gstart(j):
                    soff = pl.multiple_of(j * 128, 8)
                    b = lax.rem(j, 3)
                    pltpu.async_copy(tab.at[src_v.at[pl.ds(soff, 128)]],
                                     rows_v.at[b], sem_g.at[b])

                def gwait(j):
                    soff = pl.multiple_of(j * 128, 8)
                    b = lax.rem(j, 3)
                    pltpu.make_async_copy(
                        tab.at[src_v.at[pl.ds(soff, 128)]],
                        rows_v.at[b], sem_g.at[b]).wait()

                def sstart(j):
                    b = lax.rem(j, 3)
                    pltpu.async_copy(rows_v.at[b],
                                     acc.at[dst8.at[lax.rem(j, 8)]],
                                     sem_s.at[b], add=True)

                def swait(j):
                    b = lax.rem(j, 3)
                    pltpu.make_async_copy(rows_v.at[b],
                                          acc.at[dst8.at[lax.rem(j, 8)]],
                                          sem_s.at[b]).wait()

                @pl.when(nc >= 1)
                def _():
                    gstart(0)

                @pl.when(nc >= 2)
                def _():
                    gstart(1)

                def body(j, carry):
                    @pl.when(j >= 1)
                    def _():
                        swait(j - 1)

                    @pl.when(lax.rem(j, 8) == 0)
                    def _():
                        pltpu.sync_copy(
                            dstm.at[s * (CPT // 8) + lax.div(j, 8)], dst8)
                    gwait(j)
                    sstart(j)

                    @pl.when(j + 2 < nc)
                    def _():
                        gstart(j + 2)
                    return carry
                lax.fori_loop(0, nc, body, 0)

                @pl.when(nc >= 1)
                def _():
                    swait(nc - 1)

            @pl.when(c == 0)
            def _():
                run(tab0)

            @pl.when(c == 1)
            def _():
                run(tab1)

            plsc.subcore_barrier()
            sliced(lambda r, n: pltpu.sync_copy(
                acc.at[pl.ds(r, n)], out.at[c, pl.ds(h * NH + r, n)]))
            plsc.subcore_barrier()

        phase(tlo, thi, stt0, dtt0, ntt0, out_tt, 0)
        phase(tlo, thi, stt1, dtt1, ntt1, out_tt, 1)
        phase(rlo, rhi, srr0, drr0, nrr0, out_rr, 0)
        phase(rlo, rhi, srr1, drr1, nrr1, out_rr, 1)
        phase(rlo, rhi, srt0, drt0, nrt0, out_rt, 0)
        phase(rlo, rhi, srt1, drt1, nrt1, out_rt, 1)

    return k


@functools.cache
def _sc_partition_kernel():
    mesh = plsc.VectorSubcoreMesh(core_axis_name="c", subcore_axis_name="s")
    ECT = CPT * 128
    lsd = jax.ShapeDtypeStruct((16 * ECT,), jnp.int32)
    ncd = jax.ShapeDtypeStruct((256,), jnp.int32)

    @functools.partial(
        pl.kernel,
        out_type=tuple([lsd, lsd, ncd] * 6),
        mesh=mesh,
        compiler_params=pltpu.CompilerParams(needs_layout_passes=False),
        scratch_types=[
            pltpu.VMEM((ECT,), jnp.int32),
            pltpu.VMEM((ECT,), jnp.int32),
            pltpu.VMEM((ECT,), jnp.int32),
            pltpu.VMEM((ECT,), jnp.int32),
            pltpu.VMEM((ECT,), jnp.int32),
            pltpu.VMEM((ECT,), jnp.int32),
            pltpu.VMEM((16,), jnp.int32),
        ],
    )
    def k(stt, dtt, srr, drr, srt, drt,
          ps_tt0, pd_tt0, nc_tt0, ps_tt1, pd_tt1, nc_tt1,
          ps_rr0, pd_rr0, nc_rr0, ps_rr1, pd_rr1, nc_rr1,
          ps_rt0, pd_rt0, nc_rt0, ps_rt1, pd_rt1, nc_rt1,
          src_v, dst_v, ls0, ld0, ls1, ld1, ncb):
        c = lax.axis_index("c")
        s = lax.axis_index("s")
        lane = lax.iota(jnp.int32, 16)

        def phase(srcm, dstm, outs0, outs1):
            pltpu.sync_copy(srcm.at[pl.ds(s * ECT, ECT)], src_v)
            pltpu.sync_copy(dstm.at[pl.ds(s * ECT, ECT)], dst_v)

            def pre(i, carry):
                off = pl.multiple_of(i * 16, 8)
                iv = lane + i * 16
                ls0[pl.ds(off, 16)] = iv & 4095
                ld0[pl.ds(off, 16)] = NB + (iv & 511)
                ls1[pl.ds(off, 16)] = iv & 4095
                ld1[pl.ds(off, 16)] = NB + (iv & 511)
                return carry
            lax.fori_loop(0, ECT // 16, pre, 0)

            def body(g, carry):
                o0, o1 = carry
                off = pl.multiple_of(g * 16, 8)
                sv = src_v[pl.ds(off, 16)]
                dv = dst_v[pl.ds(off, 16)]
                m0 = dv < NB
                pos0 = o0 + plsc.cumsum(jnp.where(m0, 1, 0)) - 1
                plsc.store_scatter(ls0, [pos0], sv, mask=m0)
                plsc.store_scatter(ld0, [pos0], dv, mask=m0)
                o0 = o0 + plsc.all_reduce_population_count(m0)
                m1 = jnp.logical_and(dv >= NB, dv < N)
                pos1 = o1 + plsc.cumsum(jnp.where(m1, 1, 0)) - 1
                plsc.store_scatter(ls1, [pos1], sv, mask=m1)
                plsc.store_scatter(ld1, [pos1], dv - NB, mask=m1)
                o1 = o1 + plsc.all_reduce_population_count(m1)
                return (o0, o1)
            zero = jnp.zeros((16,), jnp.int32)
            o0, o1 = lax.fori_loop(0, ECT // 16, body, (zero, zero))

            for (ps, pd, nc), ls, ld, ov in ((outs0, ls0, ld0, o0),
                                             (outs1, ls1, ld1, o1)):
                pltpu.sync_copy(ls, ps.at[pl.ds(s * ECT, ECT)])
                pltpu.sync_copy(ld, pd.at[pl.ds(s * ECT, ECT)])
                ncb[...] = lax.shift_right_logical(ov + 127, 7)
                pltpu.sync_copy(ncb, nc.at[pl.ds(s * 16, 16)])

        @pl.when(c == 0)
        def _():
            phase(stt, dtt, (ps_tt0, pd_tt0, nc_tt0), (ps_tt1, pd_tt1, nc_tt1))
            phase(srt, drt, (ps_rt0, pd_rt0, nc_rt0), (ps_rt1, pd_rt1, nc_rt1))

        @pl.when(c == 1)
        def _():
            phase(srr, drr, (ps_rr0, pd_rr0, nc_rr0), (ps_rr1, pd_rr1, nc_rr1))

    return k


@functools.cache
def _sc_counts_kernel():
    mesh = plsc.VectorSubcoreMesh(core_axis_name="c", subcore_axis_name="s")
    csd = jax.ShapeDtypeStruct((NS, F), jnp.float32)

    @functools.partial(
        pl.kernel,
        out_type=(csd, csd, csd),
        mesh=mesh,
        compiler_params=pltpu.CompilerParams(needs_layout_passes=False),
        scratch_types=[
            pltpu.VMEM((8, 128), jnp.int32),
            pltpu.VMEM((128, F), jnp.float32),
            pltpu.VMEM((256,), jnp.int32),
            pltpu.VMEM_SHARED((NH, F), jnp.float32),
        ],
    )
    def k(dtt0, ntt0, dtt1, ntt1, drr0, nrr0, drr1, nrr1,
          drt0, nrt0, drt1, nrt1, ones, zz,
          out_tt, out_rr, out_rt, dst8, ones_v, nc_v, acc):
        c = lax.axis_index("c")
        s = lax.axis_index("s")
        r0 = s * RPTH

        def sliced(fn):
            @pl.when(s < 15)
            def _():
                fn(r0, RPTH)

            @pl.when(s == 15)
            def _():
                fn(15 * RPTH, NH - 15 * RPTH)

        pltpu.sync_copy(ones, ones_v)

        def cphase(dstm, ncm, out, h):
            sliced(lambda r, n: pltpu.sync_copy(zz.at[pl.ds(0, n)],
                                                acc.at[pl.ds(r, n)]))
            pltpu.sync_copy(ncm, nc_v)
            nc = jnp.max(nc_v[pl.ds(pl.multiple_of(s * 16, 8), 16)])
            plsc.subcore_barrier()

            def body(j, carry):
                @pl.when(lax.rem(j, 8) == 0)
                def _():
                    pltpu.sync_copy(
                        dstm.at[s * (CPT // 8) + lax.div(j, 8)], dst8)
                pltpu.sync_copy(ones_v, acc.at[dst8.at[lax.rem(j, 8)]],
                                add=True)
                return carry
            lax.fori_loop(0, nc, body, 0)
            plsc.subcore_barrier()
            sliced(lambda r, n: pltpu.sync_copy(
                acc.at[pl.ds(r, n)], out.at[pl.ds(h * NH + r, n)]))
            plsc.subcore_barrier()

        @pl.when(c == 0)
        def _():
            cphase(dtt0, ntt0, out_tt, 0)
            cphase(dtt1, ntt1, out_tt, 1)
            cphase(drt0, nrt0, out_rt, 0)
            cphase(drt1, nrt1, out_rt, 1)

        @pl.when(c == 1)
        def _():
            cphase(drr0, nrr0, out_rr, 0)
            cphase(drr1, nrr1, out_rr, 1)

    return k


def _tc_post(x, W1, Wp, g, b):
    nrows, kdim = x.shape
    R = 1000

    def body(x_ref, w1_ref, wp_ref, g_ref, b_ref, lo_ref, hi_ref):
        h = jnp.dot(x_ref[...], w1_ref[...], preferred_element_type=jnp.float32)
        h = _leaky(h)
        h = jnp.dot(h, wp_ref[...], preferred_element_type=jnp.float32)
        m = jnp.mean(h, axis=1, keepdims=True)
        v = jnp.mean((h - m) * (h - m), axis=1, keepdims=True)
        h = (h - m) * lax.rsqrt(v + 1e-5) * g_ref[...] + b_ref[...]
        h = _leaky(h)
        lo_ref[...] = h[:, :F]
        hi_ref[...] = h[:, F:]

    return pl.pallas_call(
        body,
        grid=(nrows // R,),
        in_specs=[
            pl.BlockSpec((R, kdim), lambda i: (i, 0)),
            pl.BlockSpec((kdim, HID), lambda i: (0, 0)),
            pl.BlockSpec((HID, HID), lambda i: (0, 0)),
            pl.BlockSpec((1, HID), lambda i: (0, 0)),
            pl.BlockSpec((1, HID), lambda i: (0, 0)),
        ],
        out_specs=[
            pl.BlockSpec((R, F), lambda i: (i, 0)),
            pl.BlockSpec((R, F), lambda i: (i, 0)),
        ],
        out_shape=[
            jax.ShapeDtypeStruct((nrows, F), jnp.float32),
            jax.ShapeDtypeStruct((nrows, F), jnp.float32),
        ],
    )(x, W1, Wp, g, b)


def _tc_combine_tgt(s_tt, s_rt, cnt_tt, cnt_rt, h_lo, h_hi,
                    wl_tt, wr_tt, b_tt, wl_rt, wr_rt, b_rt, final):
    R = 1000

    def body(stt_ref, srt_ref, ctt_ref, crt_ref, hlo_ref, hhi_ref,
             wltt_ref, wrtt_ref, btt_ref, wlrt_ref, wrrt_ref, brt_ref, *outs):
        ctt = jnp.maximum(ctt_ref[...], 1.0)
        crt = jnp.maximum(crt_ref[...], 1.0)
        y = jnp.dot(stt_ref[0] / ctt, wltt_ref[0:F, :],
                    preferred_element_type=jnp.float32)
        y = y + jnp.dot(stt_ref[1] / ctt, wltt_ref[F:, :],
                        preferred_element_type=jnp.float32)
        y = y + jnp.dot(srt_ref[0] / crt, wlrt_ref[0:F, :],
                        preferred_element_type=jnp.float32)
        y = y + jnp.dot(srt_ref[1] / crt, wlrt_ref[F:, :],
                        preferred_element_type=jnp.float32)
        y = y + jnp.dot(hlo_ref[...], wrtt_ref[0:F, :] + wrrt_ref[0:F, :],
                        preferred_element_type=jnp.float32)
        y = y + jnp.dot(hhi_ref[...], wrtt_ref[F:, :] + wrrt_ref[F:, :],
                        preferred_element_type=jnp.float32)
        y = y + btt_ref[...] + brt_ref[...]
        h = _leaky(0.5 * y)
        if final:
            outs[0][...] = h
        else:
            outs[0][...] = h[:, :F]
            outs[1][...] = h[:, F:]

    if final:
        out_specs = [pl.BlockSpec((R, HID), lambda i: (i, 0))]
        out_shape = [jax.ShapeDtypeStruct((N, HID), jnp.float32)]
    else:
        out_specs = [pl.BlockSpec((R, F), lambda i: (i, 0)),
                     pl.BlockSpec((R, F), lambda i: (i, 0))]
        out_shape = [jax.ShapeDtypeStruct((N, F), jnp.float32),
                     jax.ShapeDtypeStruct((N, F), jnp.float32)]

    smap = lambda i: (0, jnp.where(i < NB // R, i, i + 1), 0)
    smap2 = lambda i: (jnp.where(i < NB // R, i, i + 1), 0)
    return pl.pallas_call(
        body,
        grid=(N // R,),
        in_specs=[
            pl.BlockSpec((2, R, F), smap),
            pl.BlockSpec((2, R, F), smap),
            pl.BlockSpec((R, F), smap2),
            pl.BlockSpec((R, F), smap2),
            pl.BlockSpec((R, F), lambda i: (i, 0)),
            pl.BlockSpec((R, F), lambda i: (i, 0)),
            pl.BlockSpec((HID, HID), lambda i: (0, 0)),
            pl.BlockSpec((HID, HID), lambda i: (0, 0)),
            pl.BlockSpec((1, HID), lambda i: (0, 0)),
            pl.BlockSpec((HID, HID), lambda i: (0, 0)),
            pl.BlockSpec((HID, HID), lambda i: (0, 0)),
            pl.BlockSpec((1, HID), lambda i: (0, 0)),
        ],
        out_specs=out_specs,
        out_shape=out_shape,
    )(s_tt, s_rt, cnt_tt, cnt_rt, h_lo, h_hi,
      wl_tt, wr_tt, b_tt, wl_rt, wr_rt, b_rt)


def _tc_combine_ref(s_rr, cnt, h_lo, h_hi, wl_rr, wr_rr, b_rr, final):
    R = 1000

    def body(srr_ref, crr_ref, hlo_ref, hhi_ref,
             wl_ref, wr_ref, b_ref, *outs):
        crr = jnp.maximum(crr_ref[...], 1.0)
        y = jnp.dot(srr_ref[0] / crr, wl_ref[0:F, :],
                    preferred_element_type=jnp.float32)
        y = y + jnp.dot(srr_ref[1] / crr, wl_ref[F:, :],
                        preferred_element_type=jnp.float32)
        y = y + jnp.dot(hlo_ref[...], wr_ref[0:F, :],
                        preferred_element_type=jnp.float32)
        y = y + jnp.dot(hhi_ref[...], wr_ref[F:, :],
                        preferred_element_type=jnp.float32)
        y = y + b_ref[...]
        h = _leaky(y)
        if final:
            outs[0][...] = h
        else:
            outs[0][...] = h[:, :F]
            outs[1][...] = h[:, F:]

    if final:
        out_specs = [pl.BlockSpec((R, HID), lambda i: (i, 0))]
        out_shape = [jax.ShapeDtypeStruct((N, HID), jnp.float32)]
    else:
        out_specs = [pl.BlockSpec((R, F), lambda i: (i, 0)),
                     pl.BlockSpec((R, F), lambda i: (i, 0))]
        out_shape = [jax.ShapeDtypeStruct((N, F), jnp.float32),
                     jax.ShapeDtypeStruct((N, F), jnp.float32)]

    smap = lambda i: (0, jnp.where(i < NB // R, i, i + 1), 0)
    smap2 = lambda i: (jnp.where(i < NB // R, i, i + 1), 0)
    return pl.pallas_call(
        body,
        grid=(N // R,),
        in_specs=[
            pl.BlockSpec((2, R, F), smap),
            pl.BlockSpec((R, F), smap2),
            pl.BlockSpec((R, F), lambda i: (i, 0)),
            pl.BlockSpec((R, F), lambda i: (i, 0)),
            pl.BlockSpec((HID, HID), lambda i: (0, 0)),
            pl.BlockSpec((HID, HID), lambda i: (0, 0)),
            pl.BlockSpec((1, HID), lambda i: (0, 0)),
        ],
        out_specs=out_specs,
        out_shape=out_shape,
    )(s_rr, cnt, h_lo, h_hi, wl_rr, wr_rr, b_rr)


def _prep_edges(ei):
    e = ei.shape[1]
    pad = ECH * 128 - e
    spread = jnp.arange(pad, dtype=jnp.int32)
    src = jnp.concatenate([ei[0], spread % 4096])
    dst = jnp.concatenate([ei[1], N + (spread & 511)])
    trash = NB + (dst & 511)
    dst_h0 = jnp.where(dst < NB, dst, trash)
    dst_h1 = jnp.where(dst >= NB, dst - NB, trash)
    dst_h1 = jnp.where(dst_h1 >= NB, trash, dst_h1)
    return src, dst


def kernel(x_target, x_reference, edge_index_tt, edge_index_rr, edge_index_rt,
           params):
    p = params
    src_tt, dfl_tt = _prep_edges(edge_index_tt)
    src_rr, dfl_rr = _prep_edges(edge_index_rr)
    src_rt, dfl_rt = _prep_edges(edge_index_rt)

    zeros_f = jnp.zeros((RPTH, F), jnp.float32)
    ones_r = jnp.ones((128, F), jnp.float32)

    g2 = p['ln_g'].reshape(1, HID)
    b2 = p['ln_b'].reshape(1, HID)

    ht_lo, ht_hi = _tc_post(x_target, p['W_win'], p['W_post'], g2, b2)
    hr_lo, hr_hi = _tc_post(x_reference, p['W_exp'], p['W_post'], g2, b2)

    (ps_tt0, pd_tt0, nc_tt0, ps_tt1, pd_tt1, nc_tt1,
     ps_rr0, pd_rr0, nc_rr0, ps_rr1, pd_rr1, nc_rr1,
     ps_rt0, pd_rt0, nc_rt0, ps_rt1, pd_rt1, nc_rt1) = _sc_partition_kernel()(
        src_tt, dfl_tt, src_rr, dfl_rr, src_rt, dfl_rt)
    rs = lambda a: a.reshape(ECH // 8, 8, 128)
    pd_tt0, pd_tt1 = rs(pd_tt0), rs(pd_tt1)
    pd_rr0, pd_rr1 = rs(pd_rr0), rs(pd_rr1)
    pd_rt0, pd_rt1 = rs(pd_rt0), rs(pd_rt1)

    cnt_tt, cnt_rr, cnt_rt = _sc_counts_kernel()(
        pd_tt0, nc_tt0, pd_tt1, nc_tt1, pd_rr0, nc_rr0, pd_rr1, nc_rr1,
        pd_rt0, nc_rt0, pd_rt1, nc_rt1, ones_r, zeros_f)

    names = ('Wl_tt', 'Wr_tt', 'b_tt', 'Wl_rr', 'Wr_rr', 'b_rr',
             'Wl_rt', 'Wr_rt', 'b_rt')
    ws = {n: jnp.stack([layer[n] for layer in p['layers']]) for n in names}

    def step(carry, w):
        ht_lo, ht_hi, hr_lo, hr_hi = carry
        s_tt, s_rr, s_rt = _sc_scatter3_kernel()(
            ht_lo, ht_hi, hr_lo, hr_hi,
            ps_tt0, pd_tt0, nc_tt0, ps_tt1, pd_tt1, nc_tt1,
            ps_rr0, pd_rr0, nc_rr0, ps_rr1, pd_rr1, nc_rr1,
            ps_rt0, pd_rt0, nc_rt0, ps_rt1, pd_rt1, nc_rt1, zeros_f)
        nt_lo, nt_hi = _tc_combine_tgt(
            s_tt, s_rt, cnt_tt, cnt_rt, ht_lo, ht_hi,
            w['Wl_tt'], w['Wr_tt'], w['b_tt'].reshape(1, HID),
            w['Wl_rt'], w['Wr_rt'], w['b_rt'].reshape(1, HID), False)
        nr_lo, nr_hi = _tc_combine_ref(
            s_rr, cnt_rr, hr_lo, hr_hi,
            w['Wl_rr'], w['Wr_rr'], w['b_rr'].reshape(1, HID), False)
        return (nt_lo, nt_hi, nr_lo, nr_hi), None

    (ht_lo, ht_hi, hr_lo, hr_hi), _ = lax.scan(
        step, (ht_lo, ht_hi, hr_lo, hr_hi), ws)

    h_tgt = jnp.concatenate([ht_lo, ht_hi], axis=1)
    h_ref = jnp.concatenate([hr_lo, hr_hi], axis=1)
    return (h_tgt, h_ref)


# speedup vs baseline: 5.6790x; 1.0400x over previous
import functools

import jax
import jax.numpy as jnp
from jax import lax
from jax.experimental import pallas as pl
from jax.experimental.pallas import tpu as pltpu
from jax.experimental.pallas import tpu_sc as plsc

N = 10000
NPAD = 10240
RPT = NPAD // 16
NB = 5000
NH = 6000
RPTH = 376
NS = 2 * NH
ECH = 1280
CPT = ECH // 16
F = 128
CW = 16
HID = 256


def _leaky(x):
    return jnp.where(x >= 0.0, x, 0.2 * x)


@functools.cache
def _sc_scatter3_kernel():
    mesh = plsc.VectorSubcoreMesh(core_axis_name="c", subcore_axis_name="s")
    ssd = jax.ShapeDtypeStruct((2, NS, F), jnp.float32)
    ECT = CPT * 128

    @functools.partial(
        pl.kernel,
        out_type=(ssd, ssd, ssd),
        mesh=mesh,
        compiler_params=pltpu.CompilerParams(needs_layout_passes=False),
        scratch_types=[
            pltpu.VMEM((ECT,), jnp.int32),
            pltpu.VMEM((8, 128), jnp.int32),
            pltpu.VMEM((3, 128, F), jnp.float32),
            pltpu.VMEM((256,), jnp.int32),
            pltpu.VMEM_SHARED((NH, F), jnp.float32),
            pltpu.SemaphoreType.DMA((3,)),
            pltpu.SemaphoreType.DMA((3,)),
        ],
    )
    def k(tlo, thi, rlo, rhi,
          stt0, dtt0, ntt0, stt1, dtt1, ntt1,
          srr0, drr0, nrr0, srr1, drr1, nrr1,
          srt0, drt0, nrt0, srt1, drt1, nrt1,
          zz, out_tt, out_rr, out_rt,
          src_v, dst8, rows_v, nc_v, acc, sem_g, sem_s):
        c = lax.axis_index("c")
        s = lax.axis_index("s")
        r0 = s * RPTH

        def sliced(fn):
            @pl.when(s < 15)
            def _():
                fn(r0, RPTH)

            @pl.when(s == 15)
            def _():
                fn(15 * RPTH, NH - 15 * RPTH)

        def phase(tab0, tab1, srcm, dstm, ncm, out, h):
            sliced(lambda r, n: pltpu.sync_copy(zz.at[pl.ds(0, n)],
                                                acc.at[pl.ds(r, n)]))
            pltpu.sync_copy(srcm.at[pl.ds(s * ECT, ECT)], src_v)
            pltpu.sync_copy(ncm, nc_v)
            ncvec = nc_v[pl.ds(pl.multiple_of(s * 16, 8), 16)]
            nc = jnp.max(ncvec)
            plsc.subcore_barrier()

            def run(tab):
                def gstart(j):
                    soff = pl.multiple_of(j * 128, 8)
                    b = lax.rem(j, 3)
                    pltpu.async_copy(tab.at[src_v.at[pl.ds(soff, 128)]],
                                     rows_v.at[b], sem_g.at[b])

                def gwait(j):
                    soff = pl.multiple_of(j * 128, 8)
                    b = lax.rem(j, 3)
                    pltpu.make_async_copy(
                        tab.at[src_v.at[pl.ds(soff, 128)]],
                        rows_v.at[b], sem_g.at[b]).wait()

                def sstart(j):
                    b = lax.rem(j, 3)
                    pltpu.async_copy(rows_v.at[b],
                                     acc.at[dst8.at[lax.rem(j, 8)]],
                                     sem_s.at[b], add=True)

                def swait(j):
                    b = lax.rem(j, 3)
                    pltpu.make_async_copy(rows_v.at[b],
                                          acc.at[dst8.at[lax.rem(j, 8)]],
                                          sem_s.at[b]).wait()

                @pl.when(nc >= 1)
                def _():
                    gstart(0)

                @pl.when(nc >= 2)
                def _():
                    gstart(1)

                def body(j, carry):
                    @pl.when(j >= 1)
                    def _():
                        swait(j - 1)

                    @pl.when(lax.rem(j, 8) == 0)
                    def _():
                        pltpu.sync_copy(
                            dstm.at[s * (CPT // 8) + lax.div(j, 8)], dst8)
                    gwait(j)
                    sstart(j)

                    @pl.when(j + 2 < nc)
                    def _():
                        gstart(j + 2)
                    return carry
                lax.fori_loop(0, nc, body, 0)

                @pl.when(nc >= 1)
                def _():
                    swait(nc - 1)

            @pl.when(c == 0)
            def _():
                run(tab0)

            @pl.when(c == 1)
            def _():
                run(tab1)

            plsc.subcore_barrier()
            sliced(lambda r, n: pltpu.sync_copy(
                acc.at[pl.ds(r, n)], out.at[c, pl.ds(h * NH + r, n)]))
            plsc.subcore_barrier()

        phase(tlo, thi, stt0, dtt0, ntt0, out_tt, 0)
        phase(tlo, thi, stt1, dtt1, ntt1, out_tt, 1)
        phase(rlo, rhi, srr0, drr0, nrr0, out_rr, 0)
        phase(rlo, rhi, srr1, drr1, nrr1, out_rr, 1)
        phase(rlo, rhi, srt0, drt0, nrt0, out_rt, 0)
        phase(rlo, rhi, srt1, drt1, nrt1, out_rt, 1)

    return k


@functools.cache
def _sc_partition_kernel():
    mesh = plsc.VectorSubcoreMesh(core_axis_name="c", subcore_axis_name="s")
    ECT = CPT * 128
    lsd = jax.ShapeDtypeStruct((16 * ECT,), jnp.int32)
    ncd = jax.ShapeDtypeStruct((256,), jnp.int32)

    @functools.partial(
        pl.kernel,
        out_type=tuple([lsd, lsd, ncd] * 6),
        mesh=mesh,
        compiler_params=pltpu.CompilerParams(needs_layout_passes=False),
        scratch_types=[
            pltpu.VMEM((ECT,), jnp.int32),
            pltpu.VMEM((ECT,), jnp.int32),
            pltpu.VMEM((ECT,), jnp.int32),
            pltpu.VMEM((ECT,), jnp.int32),
            pltpu.VMEM((ECT,), jnp.int32),
            pltpu.VMEM((ECT,), jnp.int32),
            pltpu.VMEM((16,), jnp.int32),
        ],
    )
    def k(stt, dtt, srr, drr, srt, drt,
          ps_tt0, pd_tt0, nc_tt0, ps_tt1, pd_tt1, nc_tt1,
          ps_rr0, pd_rr0, nc_rr0, ps_rr1, pd_rr1, nc_rr1,
          ps_rt0, pd_rt0, nc_rt0, ps_rt1, pd_rt1, nc_rt1,
          src_v, dst_v, ls0, ld0, ls1, ld1, ncb):
        c = lax.axis_index("c")
        s = lax.axis_index("s")
        lane = lax.iota(jnp.int32, 16)

        def phase(srcm, dstm, outs0, outs1):
            pltpu.sync_copy(srcm.at[pl.ds(s * ECT, ECT)], src_v)
            pltpu.sync_copy(dstm.at[pl.ds(s * ECT, ECT)], dst_v)

            def pre(i, carry):
                off = pl.multiple_of(i * 16, 8)
                iv = lane + i * 16
                ls0[pl.ds(off, 16)] = iv & 4095
                ld0[pl.ds(off, 16)] = NB + (iv & 511)
                ls1[pl.ds(off, 16)] = iv & 4095
                ld1[pl.ds(off, 16)] = NB + (iv & 511)
                return carry
            lax.fori_loop(0, ECT // 16, pre, 0)

            def body(g, carry):
                o0, o1 = carry
                off = pl.multiple_of(g * 16, 8)
                sv = src_v[pl.ds(off, 16)]
                dv = dst_v[pl.ds(off, 16)]
                m0 = dv < NB
                pos0 = o0 + plsc.cumsum(jnp.where(m0, 1, 0)) - 1
                plsc.store_scatter(ls0, [pos0], sv, mask=m0)
                plsc.store_scatter(ld0, [pos0], dv, mask=m0)
                o0 = o0 + plsc.all_reduce_population_count(m0)
                m1 = jnp.logical_and(dv >= NB, dv < N)
                pos1 = o1 + plsc.cumsum(jnp.where(m1, 1, 0)) - 1
                plsc.store_scatter(ls1, [pos1], sv, mask=m1)
                plsc.store_scatter(ld1, [pos1], dv - NB, mask=m1)
                o1 = o1 + plsc.all_reduce_population_count(m1)
                return (o0, o1)
            zero = jnp.zeros((16,), jnp.int32)
            o0, o1 = lax.fori_loop(0, ECT // 16, body, (zero, zero))

            for (ps, pd, nc), ls, ld, ov in ((outs0, ls0, ld0, o0),
                                             (outs1, ls1, ld1, o1)):
                pltpu.sync_copy(ls, ps.at[pl.ds(s * ECT, ECT)])
                pltpu.sync_copy(ld, pd.at[pl.ds(s * ECT, ECT)])
                ncb[...] = lax.shift_right_logical(ov + 127, 7)
                pltpu.sync_copy(ncb, nc.at[pl.ds(s * 16, 16)])

        @pl.when(c == 0)
        def _():
            phase(stt, dtt, (ps_tt0, pd_tt0, nc_tt0), (ps_tt1, pd_tt1, nc_tt1))
            phase(srt, drt, (ps_rt0, pd_rt0, nc_rt0), (ps_rt1, pd_rt1, nc_rt1))

        @pl.when(c == 1)
        def _():
            phase(srr, drr, (ps_rr0, pd_rr0, nc_rr0), (ps_rr1, pd_rr1, nc_rr1))

    return k


@functools.cache
def _sc_counts_kernel():
    mesh = plsc.VectorSubcoreMesh(core_axis_name="c", subcore_axis_name="s")
    csd = jax.ShapeDtypeStruct((NS, F), jnp.float32)

    @functools.partial(
        pl.kernel,
        out_type=(csd, csd, csd),
        mesh=mesh,
        compiler_params=pltpu.CompilerParams(needs_layout_passes=False),
        scratch_types=[
            pltpu.VMEM((8, 128), jnp.int32),
            pltpu.VMEM((128, F), jnp.float32),
            pltpu.VMEM((256,), jnp.int32),
            pltpu.VMEM_SHARED((NH, F), jnp.float32),
        ],
    )
    def k(dtt0, ntt0, dtt1, ntt1, drr0, nrr0, drr1, nrr1,
          drt0, nrt0, drt1, nrt1, ones, zz,
          out_tt, out_rr, out_rt, dst8, ones_v, nc_v, acc):
        c = lax.axis_index("c")
        s = lax.axis_index("s")
        r0 = s * RPTH

        def sliced(fn):
            @pl.when(s < 15)
            def _():
                fn(r0, RPTH)

            @pl.when(s == 15)
            def _():
                fn(15 * RPTH, NH - 15 * RPTH)

        pltpu.sync_copy(ones, ones_v)

        def cphase(dstm, ncm, out, h):
            sliced(lambda r, n: pltpu.sync_copy(zz.at[pl.ds(0, n)],
                                                acc.at[pl.ds(r, n)]))
            pltpu.sync_copy(ncm, nc_v)
            nc = jnp.max(nc_v[pl.ds(pl.multiple_of(s * 16, 8), 16)])
            plsc.subcore_barrier()

            def body(j, carry):
                @pl.when(lax.rem(j, 8) == 0)
                def _():
                    pltpu.sync_copy(
                        dstm.at[s * (CPT // 8) + lax.div(j, 8)], dst8)
                pltpu.sync_copy(ones_v, acc.at[dst8.at[lax.rem(j, 8)]],
                                add=True)
                return carry
            lax.fori_loop(0, nc, body, 0)
            plsc.subcore_barrier()
            sliced(lambda r, n: pltpu.sync_copy(
                acc.at[pl.ds(r, n)], out.at[pl.ds(h * NH + r, n)]))
            plsc.subcore_barrier()

        @pl.when(c == 0)
        def _():
            cphase(dtt0, ntt0, out_tt, 0)
            cphase(dtt1, ntt1, out_tt, 1)
            cphase(drt0, nrt0, out_rt, 0)

        @pl.when(c == 1)
        def _():
            cphase(drr0, nrr0, out_rr, 0)
            cphase(drr1, nrr1, out_rr, 1)
            cphase(drt1, nrt1, out_rt, 1)

    return k


def _tc_post(x, W1, Wp, g, b):
    nrows, kdim = x.shape
    R = 1000

    def body(x_ref, w1_ref, wp_ref, g_ref, b_ref, lo_ref, hi_ref):
        h = jnp.dot(x_ref[...], w1_ref[...], preferred_element_type=jnp.float32)
        h = _leaky(h)
        h = jnp.dot(h, wp_ref[...], preferred_element_type=jnp.float32)
        m = jnp.mean(h, axis=1, keepdims=True)
        v = jnp.mean((h - m) * (h - m), axis=1, keepdims=True)
        h = (h - m) * lax.rsqrt(v + 1e-5) * g_ref[...] + b_ref[...]
        h = _leaky(h)
        lo_ref[...] = h[:, :F]
        hi_ref[...] = h[:, F:]

    return pl.pallas_call(
        body,
        grid=(nrows // R,),
        in_specs=[
            pl.BlockSpec((R, kdim), lambda i: (i, 0)),
            pl.BlockSpec((kdim, HID), lambda i: (0, 0)),
            pl.BlockSpec((HID, HID), lambda i: (0, 0)),
            pl.BlockSpec((1, HID), lambda i: (0, 0)),
            pl.BlockSpec((1, HID), lambda i: (0, 0)),
        ],
        out_specs=[
            pl.BlockSpec((R, F), lambda i: (i, 0)),
            pl.BlockSpec((R, F), lambda i: (i, 0)),
        ],
        out_shape=[
            jax.ShapeDtypeStruct((nrows, F), jnp.float32),
            jax.ShapeDtypeStruct((nrows, F), jnp.float32),
        ],
    )(x, W1, Wp, g, b)


def _tc_combine_tgt(s_tt, s_rt, cnt_tt, cnt_rt, h_lo, h_hi,
                    wl_tt, wr_tt, b_tt, wl_rt, wr_rt, b_rt, final):
    R = 1000

    def body(stt_ref, srt_ref, ctt_ref, crt_ref, hlo_ref, hhi_ref,
             wltt_ref, wrtt_ref, btt_ref, wlrt_ref, wrrt_ref, brt_ref, *outs):
        ctt = jnp.maximum(ctt_ref[...], 1.0)
        crt = jnp.maximum(crt_ref[...], 1.0)
        y = jnp.dot(stt_ref[0] / ctt, wltt_ref[0:F, :],
                    preferred_element_type=jnp.float32)
        y = y + jnp.dot(stt_ref[1] / ctt, wltt_ref[F:, :],
                        preferred_element_type=jnp.float32)
        y = y + jnp.dot(srt_ref[0] / crt, wlrt_ref[0:F, :],
                        preferred_element_type=jnp.float32)
        y = y + jnp.dot(srt_ref[1] / crt, wlrt_ref[F:, :],
                        preferred_element_type=jnp.float32)
        y = y + jnp.dot(hlo_ref[...], wrtt_ref[0:F, :] + wrrt_ref[0:F, :],
                        preferred_element_type=jnp.float32)
        y = y + jnp.dot(hhi_ref[...], wrtt_ref[F:, :] + wrrt_ref[F:, :],
                        preferred_element_type=jnp.float32)
        y = y + btt_ref[...] + brt_ref[...]
        h = _leaky(0.5 * y)
        if final:
            outs[0][...] = h
        else:
            outs[0][...] = h[:, :F]
            outs[1][...] = h[:, F:]

    if final:
        out_specs = [pl.BlockSpec((R, HID), lambda i: (i, 0))]
        out_shape = [jax.ShapeDtypeStruct((N, HID), jnp.float32)]
    else:
        out_specs = [pl.BlockSpec((R, F), lambda i: (i, 0)),
                     pl.BlockSpec((R, F), lambda i: (i, 0))]
        out_shape = [jax.ShapeDtypeStruct((N, F), jnp.float32),
                     jax.ShapeDtypeStruct((N, F), jnp.float32)]

    smap = lambda i: (0, jnp.where(i < NB // R, i, i + 1), 0)
    smap2 = lambda i: (jnp.where(i < NB // R, i, i + 1), 0)
    return pl.pallas_call(
        body,
        grid=(N // R,),
        in_specs=[
            pl.BlockSpec((2, R, F), smap),
            pl.BlockSpec((2, R, F), smap),
            pl.BlockSpec((R, F), smap2),
            pl.BlockSpec((R, F), smap2),
            pl.BlockSpec((R, F), lambda i: (i, 0)),
            pl.BlockSpec((R, F), lambda i: (i, 0)),
            pl.BlockSpec((HID, HID), lambda i: (0, 0)),
            pl.BlockSpec((HID, HID), lambda i: (0, 0)),
            pl.BlockSpec((1, HID), lambda i: (0, 0)),
            pl.BlockSpec((HID, HID), lambda i: (0, 0)),
            pl.BlockSpec((HID, HID), lambda i: (0, 0)),
            pl.BlockSpec((1, HID), lambda i: (0, 0)),
        ],
        out_specs=out_specs,
        out_shape=out_shape,
    )(s_tt, s_rt, cnt_tt, cnt_rt, h_lo, h_hi,
      wl_tt, wr_tt, b_tt, wl_rt, wr_rt, b_rt)


def _tc_combine_ref(s_rr, cnt, h_lo, h_hi, wl_rr, wr_rr, b_rr, final):
    R = 1000

    def body(srr_ref, crr_ref, hlo_ref, hhi_ref,
             wl_ref, wr_ref, b_ref, *outs):
        crr = jnp.maximum(crr_ref[...], 1.0)
        y = jnp.dot(srr_ref[0] / crr, wl_ref[0:F, :],
                    preferred_element_type=jnp.float32)
        y = y + jnp.dot(srr_ref[1] / crr, wl_ref[F:, :],
                        preferred_element_type=jnp.float32)
        y = y + jnp.dot(hlo_ref[...], wr_ref[0:F, :],
                        preferred_element_type=jnp.float32)
        y = y + jnp.dot(hhi_ref[...], wr_ref[F:, :],
                        preferred_element_type=jnp.float32)
        y = y + b_ref[...]
        h = _leaky(y)
        if final:
            outs[0][...] = h
        else:
            outs[0][...] = h[:, :F]
            outs[1][...] = h[:, F:]

    if final:
        out_specs = [pl.BlockSpec((R, HID), lambda i: (i, 0))]
        out_shape = [jax.ShapeDtypeStruct((N, HID), jnp.float32)]
    else:
        out_specs = [pl.BlockSpec((R, F), lambda i: (i, 0)),
                     pl.BlockSpec((R, F), lambda i: (i, 0))]
        out_shape = [jax.ShapeDtypeStruct((N, F), jnp.float32),
                     jax.ShapeDtypeStruct((N, F), jnp.float32)]

    smap = lambda i: (0, jnp.where(i < NB // R, i, i + 1), 0)
    smap2 = lambda i: (jnp.where(i < NB // R, i, i + 1), 0)
    return pl.pallas_call(
        body,
        grid=(N // R,),
        in_specs=[
            pl.BlockSpec((2, R, F), smap),
            pl.BlockSpec((R, F), smap2),
            pl.BlockSpec((R, F), lambda i: (i, 0)),
            pl.BlockSpec((R, F), lambda i: (i, 0)),
            pl.BlockSpec((HID, HID), lambda i: (0, 0)),
            pl.BlockSpec((HID, HID), lambda i: (0, 0)),
            pl.BlockSpec((1, HID), lambda i: (0, 0)),
        ],
        out_specs=out_specs,
        out_shape=out_shape,
    )(s_rr, cnt, h_lo, h_hi, wl_rr, wr_rr, b_rr)


def _prep_edges(ei):
    e = ei.shape[1]
    pad = ECH * 128 - e
    spread = jnp.arange(pad, dtype=jnp.int32)
    src = jnp.concatenate([ei[0], spread % 4096])
    dst = jnp.concatenate([ei[1], N + (spread & 511)])
    trash = NB + (dst & 511)
    dst_h0 = jnp.where(dst < NB, dst, trash)
    dst_h1 = jnp.where(dst >= NB, dst - NB, trash)
    dst_h1 = jnp.where(dst_h1 >= NB, trash, dst_h1)
    return src, dst


def kernel(x_target, x_reference, edge_index_tt, edge_index_rr, edge_index_rt,
           params):
    p = params
    src_tt, dfl_tt = _prep_edges(edge_index_tt)
    src_rr, dfl_rr = _prep_edges(edge_index_rr)
    src_rt, dfl_rt = _prep_edges(edge_index_rt)

    zeros_f = jnp.zeros((RPTH, F), jnp.float32)
    ones_r = jnp.ones((128, F), jnp.float32)

    g2 = p['ln_g'].reshape(1, HID)
    b2 = p['ln_b'].reshape(1, HID)

    ht_lo, ht_hi = _tc_post(x_target, p['W_win'], p['W_post'], g2, b2)
    hr_lo, hr_hi = _tc_post(x_reference, p['W_exp'], p['W_post'], g2, b2)

    (ps_tt0, pd_tt0, nc_tt0, ps_tt1, pd_tt1, nc_tt1,
     ps_rr0, pd_rr0, nc_rr0, ps_rr1, pd_rr1, nc_rr1,
     ps_rt0, pd_rt0, nc_rt0, ps_rt1, pd_rt1, nc_rt1) = _sc_partition_kernel()(
        src_tt, dfl_tt, src_rr, dfl_rr, src_rt, dfl_rt)
    rs = lambda a: a.reshape(ECH // 8, 8, 128)
    pd_tt0, pd_tt1 = rs(pd_tt0), rs(pd_tt1)
    pd_rr0, pd_rr1 = rs(pd_rr0), rs(pd_rr1)
    pd_rt0, pd_rt1 = rs(pd_rt0), rs(pd_rt1)

    cnt_tt, cnt_rr, cnt_rt = _sc_counts_kernel()(
        pd_tt0, nc_tt0, pd_tt1, nc_tt1, pd_rr0, nc_rr0, pd_rr1, nc_rr1,
        pd_rt0, nc_rt0, pd_rt1, nc_rt1, ones_r, zeros_f)

    names = ('Wl_tt', 'Wr_tt', 'b_tt', 'Wl_rr', 'Wr_rr', 'b_rr',
             'Wl_rt', 'Wr_rt', 'b_rt')
    ws = {n: jnp.stack([layer[n] for layer in p['layers']]) for n in names}

    def step(carry, w):
        ht_lo, ht_hi, hr_lo, hr_hi = carry
        s_tt, s_rr, s_rt = _sc_scatter3_kernel()(
            ht_lo, ht_hi, hr_lo, hr_hi,
            ps_tt0, pd_tt0, nc_tt0, ps_tt1, pd_tt1, nc_tt1,
            ps_rr0, pd_rr0, nc_rr0, ps_rr1, pd_rr1, nc_rr1,
            ps_rt0, pd_rt0, nc_rt0, ps_rt1, pd_rt1, nc_rt1, zeros_f)
        nt_lo, nt_hi = _tc_combine_tgt(
            s_tt, s_rt, cnt_tt, cnt_rt, ht_lo, ht_hi,
            w['Wl_tt'], w['Wr_tt'], w['b_tt'].reshape(1, HID),
            w['Wl_rt'], w['Wr_rt'], w['b_rt'].reshape(1, HID), False)
        nr_lo, nr_hi = _tc_combine_ref(
            s_rr, cnt_rr, hr_lo, hr_hi,
            w['Wl_rr'], w['Wr_rr'], w['b_rr'].reshape(1, HID), False)
        return (nt_lo, nt_hi, nr_lo, nr_hi), None

    (ht_lo, ht_hi, hr_lo, hr_hi), _ = lax.scan(
        step, (ht_lo, ht_hi, hr_lo, hr_hi), ws)

    h_tgt = jnp.concatenate([ht_lo, ht_hi], axis=1)
    h_ref = jnp.concatenate([hr_lo, hr_hi], axis=1)
    return (h_tgt, h_ref)
